# Initial kernel scaffold; baseline (speedup 1.0000x reference)
import jax
import jax.numpy as jnp
from jax.experimental import pallas as pl


def kernel(x, edge_index, edge_attr, state, batch, bond_batch, params):
    raise NotImplementedError("write your pallas kernel here")


# speedup vs baseline: 3.0801x; 3.0801x over previous
import functools

import jax
import jax.numpy as jnp
from jax import lax
from jax.experimental import pallas as pl
from jax.experimental.pallas import tpu as pltpu
from jax.experimental.pallas import tpu_sc as plsc

_N = 10000
_E = 320000
_G = 64
_F = 32

_TE = 2000
_NTE = _E // _TE
_TN = 1000
_NTN = _N // _TN

_C = 128
_NCH = _E // _C
_NW = 32
_ITER = -(-_NCH // _NW)
_ZR = _N // 16


def _sp(x):
    return jnp.maximum(x, 0.0) + jnp.log(1.0 + jnp.exp(-jnp.abs(x)))


def _mesh():
    return plsc.VectorSubcoreMesh(core_axis_name="c", subcore_axis_name="s")


_SC_PARAMS = pltpu.CompilerParams(use_tc_tiling_on_sc=False)


@functools.lru_cache(maxsize=None)
def _gather_add_fn(width):
    def body(a_hbm, b_hbm, src_hbm, dst_hbm, out_hbm, si, di, ra, rb, s1, s2):
        w = lax.axis_index("s") * 2 + lax.axis_index("c")

        def step(k, carry):
            ch = w + _NW * k

            @pl.when(ch < _NCH)
            def _():
                off = ch * _C
                pltpu.sync_copy(src_hbm.at[pl.ds(off, _C)], si)
                pltpu.sync_copy(dst_hbm.at[pl.ds(off, _C)], di)
                ca = pltpu.async_copy(a_hbm.at[si], ra, s1)
                cb = pltpu.async_copy(b_hbm.at[di], rb, s2)
                ca.wait()
                cb.wait()

                def add_row(i, c2):
                    for j in range(width // 16):
                        ra[i, pl.ds(j * 16, 16)] = (
                            ra[i, pl.ds(j * 16, 16)] + rb[i, pl.ds(j * 16, 16)]
                        )
                    return c2

                lax.fori_loop(0, _C, add_row, 0)
                pltpu.sync_copy(ra, out_hbm.at[pl.ds(off, _C)])

            return carry

        lax.fori_loop(0, _ITER, step, 0)

    return pl.kernel(
        body,
        mesh=_mesh(),
        compiler_params=_SC_PARAMS,
        out_type=jax.ShapeDtypeStruct((_E, width), jnp.float32),
        scratch_types=[
            pltpu.VMEM((_C,), jnp.int32),
            pltpu.VMEM((_C,), jnp.int32),
            pltpu.VMEM((_C, width), jnp.float32),
            pltpu.VMEM((_C, width), jnp.float32),
            pltpu.SemaphoreType.DMA,
            pltpu.SemaphoreType.DMA,
        ],
    )


@functools.lru_cache(maxsize=None)
def _scatter_fn():
    width = _F

    def body(e_hbm, dst_hbm, out_hbm, idx, ebuf, zbuf, acc):
        cid = lax.axis_index("c")
        sid = lax.axis_index("s")
        w = sid * 2 + cid

        def zrow(i, carry):
            for j in range(width // 16):
                zbuf[i, pl.ds(j * 16, 16)] = jnp.zeros((16,), jnp.float32)
            return carry

        lax.fori_loop(0, _ZR, zrow, 0)
        pltpu.sync_copy(zbuf, acc.at[pl.ds(sid * _ZR, _ZR)])
        plsc.subcore_barrier()

        def step(k, carry):
            ch = w + _NW * k

            @pl.when(ch < _NCH)
            def _():
                off = ch * _C
                pltpu.sync_copy(dst_hbm.at[pl.ds(off, _C)], idx)
                pltpu.sync_copy(e_hbm.at[pl.ds(off, _C)], ebuf)
                pltpu.sync_copy(ebuf, acc.at[idx], add=True)

            return carry

        lax.fori_loop(0, _ITER, step, 0)
        plsc.subcore_barrier()
        pltpu.sync_copy(
            acc.at[pl.ds(sid * _ZR, _ZR)],
            out_hbm.at[cid, pl.ds(sid * _ZR, _ZR)],
        )

    return pl.kernel(
        body,
        mesh=_mesh(),
        compiler_params=_SC_PARAMS,
        out_type=jax.ShapeDtypeStruct((2, _N, width), jnp.float32),
        scratch_types=[
            pltpu.VMEM((_C,), jnp.int32),
            pltpu.VMEM((_C, width), jnp.float32),
            pltpu.VMEM((_ZR, width), jnp.float32),
            pltpu.VMEM_SHARED((_N, width), jnp.float32),
        ],
    )


@functools.lru_cache(maxsize=None)
def _count_fn():
    width = 16

    def body(dst_hbm, out_hbm, idx, obuf, zbuf, acc):
        cid = lax.axis_index("c")
        sid = lax.axis_index("s")
        w = sid * 2 + cid

        def orow(i, carry):
            obuf[i, pl.ds(0, 16)] = jnp.ones((16,), jnp.float32)
            return carry

        lax.fori_loop(0, _C, orow, 0)

        def zrow(i, carry):
            zbuf[i, pl.ds(0, 16)] = jnp.zeros((16,), jnp.float32)
            return carry

        lax.fori_loop(0, _ZR, zrow, 0)
        pltpu.sync_copy(zbuf, acc.at[pl.ds(sid * _ZR, _ZR)])
        plsc.subcore_barrier()

        def step(k, carry):
            ch = w + _NW * k

            @pl.when(ch < _NCH)
            def _():
                off = ch * _C
                pltpu.sync_copy(dst_hbm.at[pl.ds(off, _C)], idx)
                pltpu.sync_copy(obuf, acc.at[idx], add=True)

            return carry

        lax.fori_loop(0, _ITER, step, 0)
        plsc.subcore_barrier()
        pltpu.sync_copy(
            acc.at[pl.ds(sid * _ZR, _ZR)],
            out_hbm.at[cid, pl.ds(sid * _ZR, _ZR)],
        )

    return pl.kernel(
        body,
        mesh=_mesh(),
        compiler_params=_SC_PARAMS,
        out_type=jax.ShapeDtypeStruct((2, _N, width), jnp.float32),
        scratch_types=[
            pltpu.VMEM((_C,), jnp.int32),
            pltpu.VMEM((_C, width), jnp.float32),
            pltpu.VMEM((_ZR, width), jnp.float32),
            pltpu.VMEM_SHARED((_N, width), jnp.float32),
        ],
    )


def _gather_rows_add(a, b, src, dst, width):
    return _gather_add_fn(width)(a, b, src, dst)


def _scatter_partials(e, dst):
    return _scatter_fn()(e, dst)


def _count_partials(dst):
    return _count_fn()(dst)


def _dot(a, b):
    return jnp.dot(a, b, preferred_element_type=jnp.float32)


def _node_proj_body(x_ref, wa_ref, wb_ref, wc_ref, a_ref, b_ref, c_ref):
    x = x_ref[...]
    a_ref[...] = _dot(x, wa_ref[...])
    b_ref[...] = _dot(x, wb_ref[...])
    c_ref[...] = _dot(x, wc_ref[...])


def _node_proj(x, w1a, w1b, wv1a):
    d = x.shape[1]
    return pl.pallas_call(
        _node_proj_body,
        grid=(_NTN,),
        in_specs=[
            pl.BlockSpec((_TN, d), lambda i: (i, 0)),
            pl.BlockSpec((d, _F), lambda i: (0, 0)),
            pl.BlockSpec((d, _F), lambda i: (0, 0)),
            pl.BlockSpec((d, _F), lambda i: (0, 0)),
        ],
        out_specs=[pl.BlockSpec((_TN, _F), lambda i: (i, 0))] * 3,
        out_shape=[jax.ShapeDtypeStruct((_N, _F), jnp.float32)] * 3,
    )(x, w1a, w1b, wv1a)


def _edge_body(g_ref, ea_ref, bb_ref, u_ref, w1c_ref, w1d_ref, b1_ref,
               w2_ref, b2_ref, w3_ref, b3_ref, e_ref, ue_ref, gc_ref):
    i = pl.program_id(0)
    bb = bb_ref[0]
    iota = lax.broadcasted_iota(jnp.int32, (1, _G), 1)
    oh = (bb == iota).astype(jnp.float32)
    u1 = _dot(u_ref[...], w1d_ref[...])
    h = (g_ref[...] + _dot(ea_ref[...], w1c_ref[...]) + _dot(oh, u1)
         + b1_ref[...])
    h = _sp(h)
    h = _sp(_dot(h, w2_ref[...]) + b2_ref[...])
    h = _sp(_dot(h, w3_ref[...]) + b3_ref[...])
    e_ref[...] = h
    part = lax.dot_general(oh, h, (((0,), (0,)), ((), ())),
                           preferred_element_type=jnp.float32)
    cnt = lax.dot_general(oh, jnp.ones((oh.shape[0], 1), jnp.float32),
                          (((0,), (0,)), ((), ())),
                          preferred_element_type=jnp.float32)

    @pl.when(i == 0)
    def _():
        ue_ref[...] = part
        gc_ref[...] = cnt

    @pl.when(i > 0)
    def _():
        ue_ref[...] = ue_ref[...] + part
        gc_ref[...] = gc_ref[...] + cnt


def _edge_mlp(g, ea, bb3, u, w1c, w1d, b1, w2, b2, w3, b3):
    ein = ea.shape[1]
    ud = u.shape[1]
    return pl.pallas_call(
        _edge_body,
        grid=(_NTE,),
        in_specs=[
            pl.BlockSpec((_TE, _F), lambda i: (i, 0)),
            pl.BlockSpec((_TE, ein), lambda i: (i, 0)),
            pl.BlockSpec((1, _TE, 1), lambda i: (i, 0, 0)),
            pl.BlockSpec((_G, ud), lambda i: (0, 0)),
            pl.BlockSpec((ein, _F), lambda i: (0, 0)),
            pl.BlockSpec((ud, _F), lambda i: (0, 0)),
            pl.BlockSpec((1, _F), lambda i: (0, 0)),
            pl.BlockSpec((_F, _F), lambda i: (0, 0)),
            pl.BlockSpec((1, _F), lambda i: (0, 0)),
            pl.BlockSpec((_F, _F), lambda i: (0, 0)),
            pl.BlockSpec((1, _F), lambda i: (0, 0)),
        ],
        out_specs=[
            pl.BlockSpec((_TE, _F), lambda i: (i, 0)),
            pl.BlockSpec((_G, _F), lambda i: (0, 0)),
            pl.BlockSpec((_G, 1), lambda i: (0, 0)),
        ],
        out_shape=[
            jax.ShapeDtypeStruct((_E, _F), jnp.float32),
            jax.ShapeDtypeStruct((_G, _F), jnp.float32),
            jax.ShapeDtypeStruct((_G, 1), jnp.float32),
        ],
    )(g, ea, bb3, u, w1c, w1d, b1, w2, b2, w3, b3)


def _node_body(vx_ref, es_ref, cn_ref, b_ref, u_ref, wv1b_ref, wv1c_ref,
               bv1_ref, wv2_ref, bv2_ref, wv3_ref, bv3_ref,
               v_ref, uv_ref, nc_ref):
    i = pl.program_id(0)
    es = es_ref[0] + es_ref[1]
    cn = cn_ref[0, :, 0:1] + cn_ref[1, :, 0:1]
    emean = es / jnp.maximum(cn, 1.0)
    bb = b_ref[0]
    iota = lax.broadcasted_iota(jnp.int32, (1, _G), 1)
    oh = (bb == iota).astype(jnp.float32)
    u1 = _dot(u_ref[...], wv1c_ref[...])
    h = vx_ref[...] + _dot(emean, wv1b_ref[...]) + _dot(oh, u1) + bv1_ref[...]
    h = _sp(h)
    h = _sp(_dot(h, wv2_ref[...]) + bv2_ref[...])
    v = _sp(_dot(h, wv3_ref[...]) + bv3_ref[...])
    v_ref[...] = v
    part = lax.dot_general(oh, v, (((0,), (0,)), ((), ())),
                           preferred_element_type=jnp.float32)
    cnt = lax.dot_general(oh, jnp.ones((oh.shape[0], 1), jnp.float32),
                          (((0,), (0,)), ((), ())),
                          preferred_element_type=jnp.float32)

    @pl.when(i == 0)
    def _():
        uv_ref[...] = part
        nc_ref[...] = cnt

    @pl.when(i > 0)
    def _():
        uv_ref[...] = uv_ref[...] + part
        nc_ref[...] = nc_ref[...] + cnt


def _node_mlp(vx, es_part, cnt_part, nb3, u, wv1b, wv1c, bv1, wv2, bv2,
              wv3, bv3):
    ud = u.shape[1]
    return pl.pallas_call(
        _node_body,
        grid=(_NTN,),
        in_specs=[
            pl.BlockSpec((_TN, _F), lambda i: (i, 0)),
            pl.BlockSpec((2, _TN, _F), lambda i: (0, i, 0)),
            pl.BlockSpec((2, _TN, 16), lambda i: (0, i, 0)),
            pl.BlockSpec((1, _TN, 1), lambda i: (i, 0, 0)),
            pl.BlockSpec((_G, ud), lambda i: (0, 0)),
            pl.BlockSpec((_F, _F), lambda i: (0, 0)),
            pl.BlockSpec((ud, _F), lambda i: (0, 0)),
            pl.BlockSpec((1, _F), lambda i: (0, 0)),
            pl.BlockSpec((_F, _F), lambda i: (0, 0)),
            pl.BlockSpec((1, _F), lambda i: (0, 0)),
            pl.BlockSpec((_F, _F), lambda i: (0, 0)),
            pl.BlockSpec((1, _F), lambda i: (0, 0)),
        ],
        out_specs=[
            pl.BlockSpec((_TN, _F), lambda i: (i, 0)),
            pl.BlockSpec((_G, _F), lambda i: (0, 0)),
            pl.BlockSpec((_G, 1), lambda i: (0, 0)),
        ],
        out_shape=[
            jax.ShapeDtypeStruct((_N, _F), jnp.float32),
            jax.ShapeDtypeStruct((_G, _F), jnp.float32),
            jax.ShapeDtypeStruct((_G, 1), jnp.float32),
        ],
    )(vx, es_part, cnt_part, nb3, u, wv1b, wv1c, bv1, wv2, bv2, wv3, bv3)


def _u_body(ue_ref, gc_ref, uv_ref, nc_ref, u_ref, wue_ref, wuv_ref, wuu_ref,
            bu1_ref, wu2_ref, bu2_ref, wu3_ref, bu3_ref, out_ref):
    ue = ue_ref[...] / jnp.maximum(gc_ref[...], 1.0)
    uv = uv_ref[...] / jnp.maximum(nc_ref[...], 1.0)
    h = _sp(_dot(ue, wue_ref[...]) + _dot(uv, wuv_ref[...])
            + _dot(u_ref[...], wuu_ref[...]) + bu1_ref[...])
    h = _sp(_dot(h, wu2_ref[...]) + bu2_ref[...])
    out_ref[...] = _sp(_dot(h, wu3_ref[...]) + bu3_ref[...])


def _u_mlp(ue_sum, gcnt, uv_sum, ncnt, u, wue, wuv, wuu, bu1, wu2, bu2,
           wu3, bu3):
    ud = u.shape[1]
    full = lambda shape: pl.BlockSpec(shape, lambda: (0,) * len(shape))
    return pl.pallas_call(
        _u_body,
        in_specs=[
            full((_G, _F)), full((_G, 1)), full((_G, _F)), full((_G, 1)),
            full((_G, ud)), full((_F, _F)), full((_F, _F)), full((ud, _F)),
            full((1, _F)), full((_F, _F)), full((1, _F)), full((_F, _F)),
            full((1, _F)),
        ],
        out_specs=full((_G, _F)),
        out_shape=jax.ShapeDtypeStruct((_G, _F), jnp.float32),
    )(ue_sum, gcnt, uv_sum, ncnt, u, wue, wuv, wuu, bu1, wu2, bu2, wu3, bu3)


def _head_proj_body(v_ref, wa_ref, wb_ref, p_ref, q_ref):
    v = v_ref[...]
    p_ref[...] = _dot(v, wa_ref[...])
    q_ref[...] = _dot(v, wb_ref[...])


def _head_proj(v, wa_pad, wb_pad):
    return pl.pallas_call(
        _head_proj_body,
        grid=(_NTN,),
        in_specs=[
            pl.BlockSpec((_TN, _F), lambda i: (i, 0)),
            pl.BlockSpec((_F, 16), lambda i: (0, 0)),
            pl.BlockSpec((_F, 16), lambda i: (0, 0)),
        ],
        out_specs=[pl.BlockSpec((_TN, 16), lambda i: (i, 0))] * 2,
        out_shape=[jax.ShapeDtypeStruct((_N, 16), jnp.float32)] * 2,
    )(v, wa_pad, wb_pad)


def _head_ij_body(r_ref, e_ref, bb_ref, u_ref, wc_ref, wd_ref, b1_ref,
                  w2_ref, b2_ref, w3_ref, b3_ref, out_ref):
    bb = bb_ref[0]
    iota = lax.broadcasted_iota(jnp.int32, (1, _G), 1)
    oh = (bb == iota).astype(jnp.float32)
    u1 = _dot(u_ref[...], wd_ref[...])
    h = (r_ref[...][:, 0:4] + _dot(e_ref[...], wc_ref[...]) + _dot(oh, u1)
         + b1_ref[...])
    h = _sp(h)
    h = _sp(_dot(h, w2_ref[...]) + b2_ref[...])
    out_ref[...] = _sp(_dot(h, w3_ref[...]) + b3_ref[...])


def _head_ij(r, e3, bb3, u, wc, wd, b1, w2, b2, w3, b3):
    return pl.pallas_call(
        _head_ij_body,
        grid=(_NTE,),
        in_specs=[
            pl.BlockSpec((_TE, 16), lambda i: (i, 0)),
            pl.BlockSpec((_TE, _F), lambda i: (i, 0)),
            pl.BlockSpec((1, _TE, 1), lambda i: (i, 0, 0)),
            pl.BlockSpec((_G, _F), lambda i: (0, 0)),
            pl.BlockSpec((_F, 4), lambda i: (0, 0)),
            pl.BlockSpec((_F, 4), lambda i: (0, 0)),
            pl.BlockSpec((1, 4), lambda i: (0, 0)),
            pl.BlockSpec((4, 4), lambda i: (0, 0)),
            pl.BlockSpec((1, 4), lambda i: (0, 0)),
            pl.BlockSpec((4, 4), lambda i: (0, 0)),
            pl.BlockSpec((1, 4), lambda i: (0, 0)),
        ],
        out_specs=pl.BlockSpec((_TE, 4), lambda i: (i, 0)),
        out_shape=jax.ShapeDtypeStruct((_E, 4), jnp.float32),
    )(r, e3, bb3, u, wc, wd, b1, w2, b2, w3, b3)


def _head_ii_body(v_ref, b_ref, u_ref, wa_ref, wb_ref, b1_ref,
                  w2_ref, b2_ref, w3_ref, b3_ref, out_ref):
    bb = b_ref[0]
    iota = lax.broadcasted_iota(jnp.int32, (1, _G), 1)
    oh = (bb == iota).astype(jnp.float32)
    u1 = _dot(u_ref[...], wb_ref[...])
    h = _sp(_dot(v_ref[...], wa_ref[...]) + _dot(oh, u1) + b1_ref[...])
    h = _sp(_dot(h, w2_ref[...]) + b2_ref[...])
    out_ref[...] = _sp(_dot(h, w3_ref[...]) + b3_ref[...])


def _head_ii(v, nb3, u, wa, wb, b1, w2, b2, w3, b3):
    return pl.pallas_call(
        _head_ii_body,
        grid=(_NTN,),
        in_specs=[
            pl.BlockSpec((_TN, _F), lambda i: (i, 0)),
            pl.BlockSpec((1, _TN, 1), lambda i: (i, 0, 0)),
            pl.BlockSpec((_G, _F), lambda i: (0, 0)),
            pl.BlockSpec((_F, 4), lambda i: (0, 0)),
            pl.BlockSpec((_F, 4), lambda i: (0, 0)),
            pl.BlockSpec((1, 4), lambda i: (0, 0)),
            pl.BlockSpec((4, 4), lambda i: (0, 0)),
            pl.BlockSpec((1, 4), lambda i: (0, 0)),
            pl.BlockSpec((4, 4), lambda i: (0, 0)),
            pl.BlockSpec((1, 4), lambda i: (0, 0)),
        ],
        out_specs=pl.BlockSpec((_TN, 4), lambda i: (i, 0)),
        out_shape=jax.ShapeDtypeStruct((_N, 4), jnp.float32),
    )(v, nb3, u, wa, wb, b1, w2, b2, w3, b3)


def kernel(x, edge_index, edge_attr, state, batch, bond_batch, params):
    src = edge_index[0]
    dst = edge_index[1]
    bb3 = bond_batch.reshape(_NTE, _TE, 1)
    nb3 = batch.reshape(_NTN, _TN, 1)

    cnt_part = _count_partials(dst)

    u = state
    xin = x
    ein = edge_attr
    for name in ("embed", "core0", "core1"):
        bp = params[name]
        (W1, b1), (W2, b2), (W3, b3) = bp["phi_e"]
        n_in = xin.shape[1]
        e_in = ein.shape[1]
        w1a = W1[:n_in]
        w1b = W1[n_in:2 * n_in]
        w1c = W1[2 * n_in:2 * n_in + e_in]
        w1d = W1[2 * n_in + e_in:]
        (Wv1, bv1), (Wv2, bv2), (Wv3, bv3) = bp["phi_v"]
        wv1a = Wv1[:n_in]
        wv1b = Wv1[n_in:n_in + _F]
        wv1c = Wv1[n_in + _F:]

        A, B, VX = _node_proj(xin, w1a, w1b, wv1a)
        Gm = _gather_rows_add(A, B, src, dst, _F)
        e_out, ue_sum, gcnt = _edge_mlp(
            Gm, ein, bb3, u, w1c, w1d, b1.reshape(1, -1), W2,
            b2.reshape(1, -1), W3, b3.reshape(1, -1))
        es_part = _scatter_partials(e_out, dst)
        v_out, uv_sum, ncnt = _node_mlp(
            VX, es_part, cnt_part, nb3, u, wv1b, wv1c, bv1.reshape(1, -1),
            Wv2, bv2.reshape(1, -1), Wv3, bv3.reshape(1, -1))

        (Wu1, bu1), (Wu2, bu2), (Wu3, bu3) = bp["phi_u"]
        wue = Wu1[:_F]
        wuv = Wu1[_F:2 * _F]
        wuu = Wu1[2 * _F:]
        u = _u_mlp(ue_sum, gcnt, uv_sum, ncnt, u, wue, wuv, wuu,
                   bu1.reshape(1, -1), Wu2, bu2.reshape(1, -1), Wu3,
                   bu3.reshape(1, -1))
        xin = v_out
        ein = e_out

    (Wi1, bi1), (Wi2, bi2), (Wi3, bi3) = params["head_ii"]
    wia = Wi1[:_F]
    wib = Wi1[_F:]
    ham_ii = _head_ii(xin, nb3, u, wia, wib, bi1.reshape(1, -1), Wi2,
                      bi2.reshape(1, -1), Wi3, bi3.reshape(1, -1))

    (Wj1, bj1), (Wj2, bj2), (Wj3, bj3) = params["head_ij"]
    wja = Wj1[:_F]
    wjb = Wj1[_F:2 * _F]
    wjc = Wj1[2 * _F:3 * _F]
    wjd = Wj1[3 * _F:]
    wja_pad = jnp.pad(wja, ((0, 0), (0, 12)))
    wjb_pad = jnp.pad(wjb, ((0, 0), (0, 12)))
    P, Q = _head_proj(xin, wja_pad, wjb_pad)
    R = _gather_rows_add(P, Q, src, dst, 16)
    ham_ij = _head_ij(R, ein, bb3, u, wjc, wjd, bj1.reshape(1, -1), Wj2,
                      bj2.reshape(1, -1), Wj3, bj3.reshape(1, -1))

    return ham_ii, ham_ij, edge_index.T


# speedup vs baseline: 3.7064x; 1.2033x over previous
import functools

import jax
import jax.numpy as jnp
from jax import lax
from jax.experimental import pallas as pl
from jax.experimental.pallas import tpu as pltpu
from jax.experimental.pallas import tpu_sc as plsc

_N = 10000
_E = 320000
_G = 64
_F = 32

_TE = 2000
_NTE = _E // _TE
_TN = 1000
_NTN = _N // _TN

_C = 128
_NCH = _E // _C
_NW = 32
_ITER = -(-_NCH // _NW)
_ZR = _N // 16


def _sp(x):
    return jnp.maximum(x, 0.0) + jnp.log(1.0 + jnp.exp(-jnp.abs(x)))


def _mesh():
    return plsc.VectorSubcoreMesh(core_axis_name="c", subcore_axis_name="s")


_SC_PARAMS = pltpu.CompilerParams(use_tc_tiling_on_sc=False)


_CM = 400
_NM = _E // _NW // _CM
_GS = 80


@functools.lru_cache(maxsize=None)
def _gather_add_fn(width):
    nsl = width // 16

    def body(a_hbm, b_hbm, src_hbm, dst_hbm, out_hbm,
             si0, si1, di0, di1, ra0, ra1, rb0, rb1,
             sI0, sI1, sG, sW0, sW1):
        si = (si0, si1)
        di = (di0, di1)
        ra = (ra0, ra1)
        rb = (rb0, rb1)
        sI = (sI0, sI1)
        sW = (sW0, sW1)
        w = lax.axis_index("s") * 2 + lax.axis_index("c")
        base = w * (_E // _NW)
        pltpu.async_copy(src_hbm.at[pl.ds(base, _CM)], si[0], sI[0])
        pltpu.async_copy(dst_hbm.at[pl.ds(base, _CM)], di[0], sI[0])

        def outer(mm, carry):
            for p in range(2):
                m = mm * 2 + p

                @pl.when(m < _NM)
                def _():
                    off = base + m * _CM
                    pltpu.make_async_copy(
                        src_hbm.at[pl.ds(0, _CM)], si[p], sI[p]).wait()
                    pltpu.make_async_copy(
                        dst_hbm.at[pl.ds(0, _CM)], di[p], sI[p]).wait()

                    @pl.when(m + 1 < _NM)
                    def _():
                        off2 = off + _CM
                        pltpu.async_copy(
                            src_hbm.at[pl.ds(off2, _CM)], si[1 - p], sI[1 - p])
                        pltpu.async_copy(
                            dst_hbm.at[pl.ds(off2, _CM)], di[1 - p], sI[1 - p])

                    @pl.when(m >= 2)
                    def _():
                        pltpu.make_async_copy(
                            ra[p], out_hbm.at[pl.ds(0, _CM)], sW[p]).wait()

                    cps = []
                    for j in range(_CM // _GS):
                        sl = pl.ds(j * _GS, _GS)
                        cps.append(pltpu.async_copy(
                            a_hbm.at[si[p].at[sl]], ra[p].at[sl], sG))
                        cps.append(pltpu.async_copy(
                            b_hbm.at[di[p].at[sl]], rb[p].at[sl], sG))
                    for cp in cps:
                        cp.wait()

                    def add8(it, c2):
                        r0 = it * 8
                        for rr in range(8):
                            for jj in range(nsl):
                                s2 = pl.ds(jj * 16, 16)
                                ra[p][r0 + rr, s2] = (
                                    ra[p][r0 + rr, s2] + rb[p][r0 + rr, s2])
                        return c2

                    lax.fori_loop(0, _CM // 8, add8, 0)
                    pltpu.async_copy(ra[p], out_hbm.at[pl.ds(off, _CM)], sW[p])

            return carry

        lax.fori_loop(0, (_NM + 1) // 2, outer, 0)
        pltpu.make_async_copy(ra[1], out_hbm.at[pl.ds(0, _CM)], sW[1]).wait()
        pltpu.make_async_copy(ra[0], out_hbm.at[pl.ds(0, _CM)], sW[0]).wait()

    return pl.kernel(
        body,
        mesh=_mesh(),
        compiler_params=_SC_PARAMS,
        out_type=jax.ShapeDtypeStruct((_E, width), jnp.float32),
        scratch_types=[
            pltpu.VMEM((_CM,), jnp.int32),
            pltpu.VMEM((_CM,), jnp.int32),
            pltpu.VMEM((_CM,), jnp.int32),
            pltpu.VMEM((_CM,), jnp.int32),
            pltpu.VMEM((_CM, width), jnp.float32),
            pltpu.VMEM((_CM, width), jnp.float32),
            pltpu.VMEM((_CM, width), jnp.float32),
            pltpu.VMEM((_CM, width), jnp.float32),
            pltpu.SemaphoreType.DMA,
            pltpu.SemaphoreType.DMA,
            pltpu.SemaphoreType.DMA,
            pltpu.SemaphoreType.DMA,
            pltpu.SemaphoreType.DMA,
        ],
    )


@functools.lru_cache(maxsize=None)
def _scatter_fn(with_count):
    width = _F

    def body(*args):
        if with_count:
            (e_hbm, dst_hbm, out_hbm, cout_hbm,
             si0, si1, eb0, eb1, ones, zb, zb2, acc, cacc,
             sI0, sI1, sE0, sE1, sO) = args
        else:
            (e_hbm, dst_hbm, out_hbm,
             si0, si1, eb0, eb1, zb, acc,
             sI0, sI1, sE0, sE1, sO) = args
        si = (si0, si1)
        eb = (eb0, eb1)
        sI = (sI0, sI1)
        sE = (sE0, sE1)
        cid = lax.axis_index("c")
        sid = lax.axis_index("s")
        w = sid * 2 + cid
        base = w * (_E // _NW)

        def zrow(i, carry):
            for jj in range(width // 16):
                zb[i, pl.ds(jj * 16, 16)] = jnp.zeros((16,), jnp.float32)
            if with_count:
                zb2[i, pl.ds(0, 16)] = jnp.zeros((16,), jnp.float32)
            return carry

        lax.fori_loop(0, _ZR, zrow, 0)
        if with_count:
            def orow(i, carry):
                ones[i, pl.ds(0, 16)] = jnp.ones((16,), jnp.float32)
                return carry

            lax.fori_loop(0, _CM, orow, 0)
        pltpu.async_copy(dst_hbm.at[pl.ds(base, _CM)], si[0], sI[0])
        pltpu.async_copy(e_hbm.at[pl.ds(base, _CM)], eb[0], sE[0])
        pltpu.sync_copy(zb, acc.at[pl.ds(sid * _ZR, _ZR)])
        if with_count:
            pltpu.sync_copy(zb2, cacc.at[pl.ds(sid * _ZR, _ZR)])
        plsc.subcore_barrier()

        def outer(mm, carry):
            for p in range(2):
                m = mm * 2 + p

                @pl.when(m < _NM)
                def _():
                    off = base + m * _CM
                    pltpu.make_async_copy(
                        dst_hbm.at[pl.ds(0, _CM)], si[p], sI[p]).wait()
                    pltpu.make_async_copy(
                        e_hbm.at[pl.ds(0, _CM)], eb[p], sE[p]).wait()

                    @pl.when(m + 1 < _NM)
                    def _():
                        off2 = off + _CM
                        pltpu.async_copy(
                            dst_hbm.at[pl.ds(off2, _CM)], si[1 - p], sI[1 - p])
                        pltpu.async_copy(
                            e_hbm.at[pl.ds(off2, _CM)], eb[1 - p], sE[1 - p])

                    pltpu.sync_copy(eb[p], acc.at[si[p]], add=True)
                    if with_count:
                        pltpu.sync_copy(ones, cacc.at[si[p]], add=True)

            return carry

        lax.fori_loop(0, (_NM + 1) // 2, outer, 0)
        plsc.subcore_barrier()
        rows = pl.ds(sid * _ZR, _ZR)
        cps = [pltpu.async_copy(acc.at[rows], out_hbm.at[cid, rows], sO)]
        if with_count:
            cps.append(pltpu.async_copy(cacc.at[rows], cout_hbm.at[cid, rows], sO))
        for cp in cps:
            cp.wait()

    out_type = [jax.ShapeDtypeStruct((2, _N, width), jnp.float32)]
    scratch = [
        pltpu.VMEM((_CM,), jnp.int32),
        pltpu.VMEM((_CM,), jnp.int32),
        pltpu.VMEM((_CM, width), jnp.float32),
        pltpu.VMEM((_CM, width), jnp.float32),
    ]
    if with_count:
        out_type.append(jax.ShapeDtypeStruct((2, _N, 16), jnp.float32))
        scratch.append(pltpu.VMEM((_CM, 16), jnp.float32))
    scratch.append(pltpu.VMEM((_ZR, width), jnp.float32))
    if with_count:
        scratch.append(pltpu.VMEM((_ZR, 16), jnp.float32))
    scratch.append(pltpu.VMEM_SHARED((_N, width), jnp.float32))
    if with_count:
        scratch.append(pltpu.VMEM_SHARED((_N, 16), jnp.float32))
    scratch += [pltpu.SemaphoreType.DMA] * 5
    return pl.kernel(
        body,
        mesh=_mesh(),
        compiler_params=_SC_PARAMS,
        out_type=out_type if with_count else out_type[0],
        scratch_types=scratch,
    )


def _gather_rows_add(a, b, src, dst, width):
    return _gather_add_fn(width)(a, b, src, dst)


def _scatter_partials(e, dst):
    return _scatter_fn(False)(e, dst)


def _scatter_count_partials(e, dst):
    return _scatter_fn(True)(e, dst)


def _dot(a, b):
    return jnp.dot(a, b, preferred_element_type=jnp.float32)


def _node_proj_body(x_ref, wa_ref, wb_ref, wc_ref, a_ref, b_ref, c_ref):
    x = x_ref[...]
    a_ref[...] = _dot(x, wa_ref[...])
    b_ref[...] = _dot(x, wb_ref[...])
    c_ref[...] = _dot(x, wc_ref[...])


def _node_proj(x, w1a, w1b, wv1a):
    d = x.shape[1]
    return pl.pallas_call(
        _node_proj_body,
        grid=(_NTN,),
        in_specs=[
            pl.BlockSpec((_TN, d), lambda i: (i, 0)),
            pl.BlockSpec((d, _F), lambda i: (0, 0)),
            pl.BlockSpec((d, _F), lambda i: (0, 0)),
            pl.BlockSpec((d, _F), lambda i: (0, 0)),
        ],
        out_specs=[pl.BlockSpec((_TN, _F), lambda i: (i, 0))] * 3,
        out_shape=[jax.ShapeDtypeStruct((_N, _F), jnp.float32)] * 3,
    )(x, w1a, w1b, wv1a)


def _edge_body(g_ref, ea_ref, bb_ref, u_ref, w1c_ref, w1d_ref, b1_ref,
               w2_ref, b2_ref, w3_ref, b3_ref, e_ref, ue_ref, gc_ref):
    i = pl.program_id(0)
    bb = bb_ref[0]
    iota = lax.broadcasted_iota(jnp.int32, (1, _G), 1)
    oh = (bb == iota).astype(jnp.float32)
    u1 = _dot(u_ref[...], w1d_ref[...])
    h = (g_ref[...] + _dot(ea_ref[...], w1c_ref[...]) + _dot(oh, u1)
         + b1_ref[...])
    h = _sp(h)
    h = _sp(_dot(h, w2_ref[...]) + b2_ref[...])
    h = _sp(_dot(h, w3_ref[...]) + b3_ref[...])
    e_ref[...] = h
    part = lax.dot_general(oh, h, (((0,), (0,)), ((), ())),
                           preferred_element_type=jnp.float32)
    cnt = lax.dot_general(oh, jnp.ones((oh.shape[0], 1), jnp.float32),
                          (((0,), (0,)), ((), ())),
                          preferred_element_type=jnp.float32)

    @pl.when(i == 0)
    def _():
        ue_ref[...] = part
        gc_ref[...] = cnt

    @pl.when(i > 0)
    def _():
        ue_ref[...] = ue_ref[...] + part
        gc_ref[...] = gc_ref[...] + cnt


def _edge_mlp(g, ea, bb3, u, w1c, w1d, b1, w2, b2, w3, b3):
    ein = ea.shape[1]
    ud = u.shape[1]
    return pl.pallas_call(
        _edge_body,
        grid=(_NTE,),
        in_specs=[
            pl.BlockSpec((_TE, _F), lambda i: (i, 0)),
            pl.BlockSpec((_TE, ein), lambda i: (i, 0)),
            pl.BlockSpec((1, _TE, 1), lambda i: (i, 0, 0)),
            pl.BlockSpec((_G, ud), lambda i: (0, 0)),
            pl.BlockSpec((ein, _F), lambda i: (0, 0)),
            pl.BlockSpec((ud, _F), lambda i: (0, 0)),
            pl.BlockSpec((1, _F), lambda i: (0, 0)),
            pl.BlockSpec((_F, _F), lambda i: (0, 0)),
            pl.BlockSpec((1, _F), lambda i: (0, 0)),
            pl.BlockSpec((_F, _F), lambda i: (0, 0)),
            pl.BlockSpec((1, _F), lambda i: (0, 0)),
        ],
        out_specs=[
            pl.BlockSpec((_TE, _F), lambda i: (i, 0)),
            pl.BlockSpec((_G, _F), lambda i: (0, 0)),
            pl.BlockSpec((_G, 1), lambda i: (0, 0)),
        ],
        out_shape=[
            jax.ShapeDtypeStruct((_E, _F), jnp.float32),
            jax.ShapeDtypeStruct((_G, _F), jnp.float32),
            jax.ShapeDtypeStruct((_G, 1), jnp.float32),
        ],
    )(g, ea, bb3, u, w1c, w1d, b1, w2, b2, w3, b3)


def _node_body(vx_ref, es_ref, cn_ref, b_ref, u_ref, wv1b_ref, wv1c_ref,
               bv1_ref, wv2_ref, bv2_ref, wv3_ref, bv3_ref,
               v_ref, uv_ref, nc_ref):
    i = pl.program_id(0)
    es = es_ref[0] + es_ref[1]
    cn = cn_ref[0, :, 0:1] + cn_ref[1, :, 0:1]
    emean = es / jnp.maximum(cn, 1.0)
    bb = b_ref[0]
    iota = lax.broadcasted_iota(jnp.int32, (1, _G), 1)
    oh = (bb == iota).astype(jnp.float32)
    u1 = _dot(u_ref[...], wv1c_ref[...])
    h = vx_ref[...] + _dot(emean, wv1b_ref[...]) + _dot(oh, u1) + bv1_ref[...]
    h = _sp(h)
    h = _sp(_dot(h, wv2_ref[...]) + bv2_ref[...])
    v = _sp(_dot(h, wv3_ref[...]) + bv3_ref[...])
    v_ref[...] = v
    part = lax.dot_general(oh, v, (((0,), (0,)), ((), ())),
                           preferred_element_type=jnp.float32)
    cnt = lax.dot_general(oh, jnp.ones((oh.shape[0], 1), jnp.float32),
                          (((0,), (0,)), ((), ())),
                          preferred_element_type=jnp.float32)

    @pl.when(i == 0)
    def _():
        uv_ref[...] = part
        nc_ref[...] = cnt

    @pl.when(i > 0)
    def _():
        uv_ref[...] = uv_ref[...] + part
        nc_ref[...] = nc_ref[...] + cnt


def _node_mlp(vx, es_part, cnt_part, nb3, u, wv1b, wv1c, bv1, wv2, bv2,
              wv3, bv3):
    ud = u.shape[1]
    return pl.pallas_call(
        _node_body,
        grid=(_NTN,),
        in_specs=[
            pl.BlockSpec((_TN, _F), lambda i: (i, 0)),
            pl.BlockSpec((2, _TN, _F), lambda i: (0, i, 0)),
            pl.BlockSpec((2, _TN, 16), lambda i: (0, i, 0)),
            pl.BlockSpec((1, _TN, 1), lambda i: (i, 0, 0)),
            pl.BlockSpec((_G, ud), lambda i: (0, 0)),
            pl.BlockSpec((_F, _F), lambda i: (0, 0)),
            pl.BlockSpec((ud, _F), lambda i: (0, 0)),
            pl.BlockSpec((1, _F), lambda i: (0, 0)),
            pl.BlockSpec((_F, _F), lambda i: (0, 0)),
            pl.BlockSpec((1, _F), lambda i: (0, 0)),
            pl.BlockSpec((_F, _F), lambda i: (0, 0)),
            pl.BlockSpec((1, _F), lambda i: (0, 0)),
        ],
        out_specs=[
            pl.BlockSpec((_TN, _F), lambda i: (i, 0)),
            pl.BlockSpec((_G, _F), lambda i: (0, 0)),
            pl.BlockSpec((_G, 1), lambda i: (0, 0)),
        ],
        out_shape=[
            jax.ShapeDtypeStruct((_N, _F), jnp.float32),
            jax.ShapeDtypeStruct((_G, _F), jnp.float32),
            jax.ShapeDtypeStruct((_G, 1), jnp.float32),
        ],
    )(vx, es_part, cnt_part, nb3, u, wv1b, wv1c, bv1, wv2, bv2, wv3, bv3)


def _u_body(ue_ref, gc_ref, uv_ref, nc_ref, u_ref, wue_ref, wuv_ref, wuu_ref,
            bu1_ref, wu2_ref, bu2_ref, wu3_ref, bu3_ref, out_ref):
    ue = ue_ref[...] / jnp.maximum(gc_ref[...], 1.0)
    uv = uv_ref[...] / jnp.maximum(nc_ref[...], 1.0)
    h = _sp(_dot(ue, wue_ref[...]) + _dot(uv, wuv_ref[...])
            + _dot(u_ref[...], wuu_ref[...]) + bu1_ref[...])
    h = _sp(_dot(h, wu2_ref[...]) + bu2_ref[...])
    out_ref[...] = _sp(_dot(h, wu3_ref[...]) + bu3_ref[...])


def _u_mlp(ue_sum, gcnt, uv_sum, ncnt, u, wue, wuv, wuu, bu1, wu2, bu2,
           wu3, bu3):
    ud = u.shape[1]
    full = lambda shape: pl.BlockSpec(shape, lambda: (0,) * len(shape))
    return pl.pallas_call(
        _u_body,
        in_specs=[
            full((_G, _F)), full((_G, 1)), full((_G, _F)), full((_G, 1)),
            full((_G, ud)), full((_F, _F)), full((_F, _F)), full((ud, _F)),
            full((1, _F)), full((_F, _F)), full((1, _F)), full((_F, _F)),
            full((1, _F)),
        ],
        out_specs=full((_G, _F)),
        out_shape=jax.ShapeDtypeStruct((_G, _F), jnp.float32),
    )(ue_sum, gcnt, uv_sum, ncnt, u, wue, wuv, wuu, bu1, wu2, bu2, wu3, bu3)


def _head_proj_body(v_ref, wa_ref, wb_ref, p_ref, q_ref):
    v = v_ref[...]
    p_ref[...] = _dot(v, wa_ref[...])
    q_ref[...] = _dot(v, wb_ref[...])


def _head_proj(v, wa_pad, wb_pad):
    return pl.pallas_call(
        _head_proj_body,
        grid=(_NTN,),
        in_specs=[
            pl.BlockSpec((_TN, _F), lambda i: (i, 0)),
            pl.BlockSpec((_F, 16), lambda i: (0, 0)),
            pl.BlockSpec((_F, 16), lambda i: (0, 0)),
        ],
        out_specs=[pl.BlockSpec((_TN, 16), lambda i: (i, 0))] * 2,
        out_shape=[jax.ShapeDtypeStruct((_N, 16), jnp.float32)] * 2,
    )(v, wa_pad, wb_pad)


def _head_ij_body(r_ref, e_ref, bb_ref, u_ref, wc_ref, wd_ref, b1_ref,
                  w2_ref, b2_ref, w3_ref, b3_ref, out_ref):
    bb = bb_ref[0]
    iota = lax.broadcasted_iota(jnp.int32, (1, _G), 1)
    oh = (bb == iota).astype(jnp.float32)
    u1 = _dot(u_ref[...], wd_ref[...])
    h = (r_ref[...][:, 0:4] + _dot(e_ref[...], wc_ref[...]) + _dot(oh, u1)
         + b1_ref[...])
    h = _sp(h)
    h = _sp(_dot(h, w2_ref[...]) + b2_ref[...])
    out_ref[...] = _sp(_dot(h, w3_ref[...]) + b3_ref[...])


def _head_ij(r, e3, bb3, u, wc, wd, b1, w2, b2, w3, b3):
    return pl.pallas_call(
        _head_ij_body,
        grid=(_NTE,),
        in_specs=[
            pl.BlockSpec((_TE, 16), lambda i: (i, 0)),
            pl.BlockSpec((_TE, _F), lambda i: (i, 0)),
            pl.BlockSpec((1, _TE, 1), lambda i: (i, 0, 0)),
            pl.BlockSpec((_G, _F), lambda i: (0, 0)),
            pl.BlockSpec((_F, 4), lambda i: (0, 0)),
            pl.BlockSpec((_F, 4), lambda i: (0, 0)),
            pl.BlockSpec((1, 4), lambda i: (0, 0)),
            pl.BlockSpec((4, 4), lambda i: (0, 0)),
            pl.BlockSpec((1, 4), lambda i: (0, 0)),
            pl.BlockSpec((4, 4), lambda i: (0, 0)),
            pl.BlockSpec((1, 4), lambda i: (0, 0)),
        ],
        out_specs=pl.BlockSpec((_TE, 4), lambda i: (i, 0)),
        out_shape=jax.ShapeDtypeStruct((_E, 4), jnp.float32),
    )(r, e3, bb3, u, wc, wd, b1, w2, b2, w3, b3)


def _head_ii_body(v_ref, b_ref, u_ref, wa_ref, wb_ref, b1_ref,
                  w2_ref, b2_ref, w3_ref, b3_ref, out_ref):
    bb = b_ref[0]
    iota = lax.broadcasted_iota(jnp.int32, (1, _G), 1)
    oh = (bb == iota).astype(jnp.float32)
    u1 = _dot(u_ref[...], wb_ref[...])
    h = _sp(_dot(v_ref[...], wa_ref[...]) + _dot(oh, u1) + b1_ref[...])
    h = _sp(_dot(h, w2_ref[...]) + b2_ref[...])
    out_ref[...] = _sp(_dot(h, w3_ref[...]) + b3_ref[...])


def _head_ii(v, nb3, u, wa, wb, b1, w2, b2, w3, b3):
    return pl.pallas_call(
        _head_ii_body,
        grid=(_NTN,),
        in_specs=[
            pl.BlockSpec((_TN, _F), lambda i: (i, 0)),
            pl.BlockSpec((1, _TN, 1), lambda i: (i, 0, 0)),
            pl.BlockSpec((_G, _F), lambda i: (0, 0)),
            pl.BlockSpec((_F, 4), lambda i: (0, 0)),
            pl.BlockSpec((_F, 4), lambda i: (0, 0)),
            pl.BlockSpec((1, 4), lambda i: (0, 0)),
            pl.BlockSpec((4, 4), lambda i: (0, 0)),
            pl.BlockSpec((1, 4), lambda i: (0, 0)),
            pl.BlockSpec((4, 4), lambda i: (0, 0)),
            pl.BlockSpec((1, 4), lambda i: (0, 0)),
        ],
        out_specs=pl.BlockSpec((_TN, 4), lambda i: (i, 0)),
        out_shape=jax.ShapeDtypeStruct((_N, 4), jnp.float32),
    )(v, nb3, u, wa, wb, b1, w2, b2, w3, b3)


def kernel(x, edge_index, edge_attr, state, batch, bond_batch, params):
    src = edge_index[0]
    dst = edge_index[1]
    bb3 = bond_batch.reshape(_NTE, _TE, 1)
    nb3 = batch.reshape(_NTN, _TN, 1)

    cnt_part = None
    u = state
    xin = x
    ein = edge_attr
    for name in ("embed", "core0", "core1"):
        bp = params[name]
        (W1, b1), (W2, b2), (W3, b3) = bp["phi_e"]
        n_in = xin.shape[1]
        e_in = ein.shape[1]
        w1a = W1[:n_in]
        w1b = W1[n_in:2 * n_in]
        w1c = W1[2 * n_in:2 * n_in + e_in]
        w1d = W1[2 * n_in + e_in:]
        (Wv1, bv1), (Wv2, bv2), (Wv3, bv3) = bp["phi_v"]
        wv1a = Wv1[:n_in]
        wv1b = Wv1[n_in:n_in + _F]
        wv1c = Wv1[n_in + _F:]

        A, B, VX = _node_proj(xin, w1a, w1b, wv1a)
        Gm = _gather_rows_add(A, B, src, dst, _F)
        e_out, ue_sum, gcnt = _edge_mlp(
            Gm, ein, bb3, u, w1c, w1d, b1.reshape(1, -1), W2,
            b2.reshape(1, -1), W3, b3.reshape(1, -1))
        if cnt_part is None:
            es_part, cnt_part = _scatter_count_partials(e_out, dst)
        else:
            es_part = _scatter_partials(e_out, dst)
        v_out, uv_sum, ncnt = _node_mlp(
            VX, es_part, cnt_part, nb3, u, wv1b, wv1c, bv1.reshape(1, -1),
            Wv2, bv2.reshape(1, -1), Wv3, bv3.reshape(1, -1))

        (Wu1, bu1), (Wu2, bu2), (Wu3, bu3) = bp["phi_u"]
        wue = Wu1[:_F]
        wuv = Wu1[_F:2 * _F]
        wuu = Wu1[2 * _F:]
        u = _u_mlp(ue_sum, gcnt, uv_sum, ncnt, u, wue, wuv, wuu,
                   bu1.reshape(1, -1), Wu2, bu2.reshape(1, -1), Wu3,
                   bu3.reshape(1, -1))
        xin = v_out
        ein = e_out

    (Wi1, bi1), (Wi2, bi2), (Wi3, bi3) = params["head_ii"]
    wia = Wi1[:_F]
    wib = Wi1[_F:]
    ham_ii = _head_ii(xin, nb3, u, wia, wib, bi1.reshape(1, -1), Wi2,
                      bi2.reshape(1, -1), Wi3, bi3.reshape(1, -1))

    (Wj1, bj1), (Wj2, bj2), (Wj3, bj3) = params["head_ij"]
    wja = Wj1[:_F]
    wjb = Wj1[_F:2 * _F]
    wjc = Wj1[2 * _F:3 * _F]
    wjd = Wj1[3 * _F:]
    wja_pad = jnp.pad(wja, ((0, 0), (0, 12)))
    wjb_pad = jnp.pad(wjb, ((0, 0), (0, 12)))
    P, Q = _head_proj(xin, wja_pad, wjb_pad)
    R = _gather_rows_add(P, Q, src, dst, 16)
    ham_ij = _head_ij(R, ein, bb3, u, wjc, wjd, bj1.reshape(1, -1), Wj2,
                      bj2.reshape(1, -1), Wj3, bj3.reshape(1, -1))

    return ham_ii, ham_ij, edge_index.T


# speedup vs baseline: 7.0335x; 1.8977x over previous
import functools

import jax
import jax.numpy as jnp
from jax import lax
from jax.experimental import pallas as pl
from jax.experimental.pallas import tpu as pltpu
from jax.experimental.pallas import tpu_sc as plsc

_N = 10000
_E = 320000
_G = 64
_F = 32

_TE4 = 1000
_NTE4 = _E // 4 // _TE4
_TN4 = 2500
_NTN4 = _N // 4 // _TN4

_NW = 32
_ZR = _N // 16


def _sp(x):
    return jnp.maximum(x, 0.0) + jnp.log(1.0 + jnp.exp(-jnp.abs(x)))


def _mesh():
    return plsc.VectorSubcoreMesh(core_axis_name="c", subcore_axis_name="s")


_SC_PARAMS = pltpu.CompilerParams(use_tc_tiling_on_sc=False)


_CM = 400
_NM = _E // _NW // _CM
_GS = 80


@functools.lru_cache(maxsize=None)
def _gather_add_fn(width):
    nsl = width // 16

    def body(a_hbm, b_hbm, src_hbm, dst_hbm, out_hbm,
             si0, si1, di0, di1, ra0, ra1, rb0, rb1,
             sI0, sI1, sG, sW0, sW1):
        si = (si0, si1)
        di = (di0, di1)
        ra = (ra0, ra1)
        rb = (rb0, rb1)
        sI = (sI0, sI1)
        sW = (sW0, sW1)
        w = lax.axis_index("s") * 2 + lax.axis_index("c")
        base = w * (_E // _NW)
        pltpu.async_copy(src_hbm.at[pl.ds(base, _CM)], si[0], sI[0])
        pltpu.async_copy(dst_hbm.at[pl.ds(base, _CM)], di[0], sI[0])

        def outer(mm, carry):
            for p in range(2):
                m = mm * 2 + p

                @pl.when(m < _NM)
                def _():
                    off = base + m * _CM
                    pltpu.make_async_copy(
                        src_hbm.at[pl.ds(0, _CM)], si[p], sI[p]).wait()
                    pltpu.make_async_copy(
                        dst_hbm.at[pl.ds(0, _CM)], di[p], sI[p]).wait()

                    @pl.when(m + 1 < _NM)
                    def _():
                        off2 = off + _CM
                        pltpu.async_copy(
                            src_hbm.at[pl.ds(off2, _CM)], si[1 - p], sI[1 - p])
                        pltpu.async_copy(
                            dst_hbm.at[pl.ds(off2, _CM)], di[1 - p], sI[1 - p])

                    @pl.when(m >= 2)
                    def _():
                        pltpu.make_async_copy(
                            ra[p], out_hbm.at[pl.ds(0, _CM)], sW[p]).wait()

                    cps = []
                    for j in range(_CM // _GS):
                        sl = pl.ds(j * _GS, _GS)
                        cps.append(pltpu.async_copy(
                            a_hbm.at[si[p].at[sl]], ra[p].at[sl], sG))
                        cps.append(pltpu.async_copy(
                            b_hbm.at[di[p].at[sl]], rb[p].at[sl], sG))
                    for cp in cps:
                        cp.wait()

                    def add8(it, c2):
                        r0 = it * 8
                        for rr in range(8):
                            for jj in range(nsl):
                                s2 = pl.ds(jj * 16, 16)
                                ra[p][r0 + rr, s2] = (
                                    ra[p][r0 + rr, s2] + rb[p][r0 + rr, s2])
                        return c2

                    lax.fori_loop(0, _CM // 8, add8, 0)
                    pltpu.async_copy(ra[p], out_hbm.at[pl.ds(off, _CM)], sW[p])

            return carry

        lax.fori_loop(0, (_NM + 1) // 2, outer, 0)
        pltpu.make_async_copy(ra[1], out_hbm.at[pl.ds(0, _CM)], sW[1]).wait()
        pltpu.make_async_copy(ra[0], out_hbm.at[pl.ds(0, _CM)], sW[0]).wait()

    return pl.kernel(
        body,
        mesh=_mesh(),
        compiler_params=_SC_PARAMS,
        out_type=jax.ShapeDtypeStruct((_E, width), jnp.float32),
        scratch_types=[
            pltpu.VMEM((_CM,), jnp.int32),
            pltpu.VMEM((_CM,), jnp.int32),
            pltpu.VMEM((_CM,), jnp.int32),
            pltpu.VMEM((_CM,), jnp.int32),
            pltpu.VMEM((_CM, width), jnp.float32),
            pltpu.VMEM((_CM, width), jnp.float32),
            pltpu.VMEM((_CM, width), jnp.float32),
            pltpu.VMEM((_CM, width), jnp.float32),
            pltpu.SemaphoreType.DMA,
            pltpu.SemaphoreType.DMA,
            pltpu.SemaphoreType.DMA,
            pltpu.SemaphoreType.DMA,
            pltpu.SemaphoreType.DMA,
        ],
    )


@functools.lru_cache(maxsize=None)
def _scatter_fn(with_count):
    width = _F

    def body(*args):
        if with_count:
            (e_hbm, dst_hbm, out_hbm, cout_hbm,
             si0, si1, eb0, eb1, ones, zb, zb2, acc, cacc,
             sI0, sI1, sE0, sE1, sO) = args
        else:
            (e_hbm, dst_hbm, out_hbm,
             si0, si1, eb0, eb1, zb, acc,
             sI0, sI1, sE0, sE1, sO) = args
        si = (si0, si1)
        eb = (eb0, eb1)
        sI = (sI0, sI1)
        sE = (sE0, sE1)
        cid = lax.axis_index("c")
        sid = lax.axis_index("s")
        w = sid * 2 + cid
        base = w * (_E // _NW)

        def zrow(i, carry):
            for jj in range(width // 16):
                zb[i, pl.ds(jj * 16, 16)] = jnp.zeros((16,), jnp.float32)
            if with_count:
                zb2[i, pl.ds(0, 16)] = jnp.zeros((16,), jnp.float32)
            return carry

        lax.fori_loop(0, _ZR, zrow, 0)
        if with_count:
            def orow(i, carry):
                ones[i, pl.ds(0, 16)] = jnp.ones((16,), jnp.float32)
                return carry

            lax.fori_loop(0, _CM, orow, 0)
        pltpu.async_copy(dst_hbm.at[pl.ds(base, _CM)], si[0], sI[0])
        pltpu.async_copy(e_hbm.at[pl.ds(base, _CM)], eb[0], sE[0])
        pltpu.sync_copy(zb, acc.at[pl.ds(sid * _ZR, _ZR)])
        if with_count:
            pltpu.sync_copy(zb2, cacc.at[pl.ds(sid * _ZR, _ZR)])
        plsc.subcore_barrier()

        def outer(mm, carry):
            for p in range(2):
                m = mm * 2 + p

                @pl.when(m < _NM)
                def _():
                    off = base + m * _CM
                    pltpu.make_async_copy(
                        dst_hbm.at[pl.ds(0, _CM)], si[p], sI[p]).wait()
                    pltpu.make_async_copy(
                        e_hbm.at[pl.ds(0, _CM)], eb[p], sE[p]).wait()

                    @pl.when(m + 1 < _NM)
                    def _():
                        off2 = off + _CM
                        pltpu.async_copy(
                            dst_hbm.at[pl.ds(off2, _CM)], si[1 - p], sI[1 - p])
                        pltpu.async_copy(
                            e_hbm.at[pl.ds(off2, _CM)], eb[1 - p], sE[1 - p])

                    pltpu.sync_copy(eb[p], acc.at[si[p]], add=True)
                    if with_count:
                        pltpu.sync_copy(ones, cacc.at[si[p]], add=True)

            return carry

        lax.fori_loop(0, (_NM + 1) // 2, outer, 0)
        plsc.subcore_barrier()
        rows = pl.ds(sid * _ZR, _ZR)
        cps = [pltpu.async_copy(acc.at[rows], out_hbm.at[cid, rows], sO)]
        if with_count:
            cps.append(pltpu.async_copy(cacc.at[rows], cout_hbm.at[cid, rows],
                                        sO))
        for cp in cps:
            cp.wait()

    out_type = [jax.ShapeDtypeStruct((2, _N, width), jnp.float32)]
    scratch = [
        pltpu.VMEM((_CM,), jnp.int32),
        pltpu.VMEM((_CM,), jnp.int32),
        pltpu.VMEM((_CM, width), jnp.float32),
        pltpu.VMEM((_CM, width), jnp.float32),
    ]
    if with_count:
        out_type.append(jax.ShapeDtypeStruct((2, _N, 16), jnp.float32))
        scratch.append(pltpu.VMEM((_CM, 16), jnp.float32))
    scratch.append(pltpu.VMEM((_ZR, width), jnp.float32))
    if with_count:
        scratch.append(pltpu.VMEM((_ZR, 16), jnp.float32))
    scratch.append(pltpu.VMEM_SHARED((_N, width), jnp.float32))
    if with_count:
        scratch.append(pltpu.VMEM_SHARED((_N, 16), jnp.float32))
    scratch += [pltpu.SemaphoreType.DMA] * 5
    return pl.kernel(
        body,
        mesh=_mesh(),
        compiler_params=_SC_PARAMS,
        out_type=out_type if with_count else out_type[0],
        scratch_types=scratch,
    )


def _gather_rows_add(a, b, src, dst, width):
    return _gather_add_fn(width)(a, b, src, dst)


def _scatter_partials(e, dst):
    return _scatter_fn(False)(e, dst)


def _scatter_count_partials(e, dst):
    return _scatter_fn(True)(e, dst)


def _dot(a, b):
    return jnp.dot(a, b, preferred_element_type=jnp.float32)


def _dg0(a, b):
    return lax.dot_general(a, b, (((0,), (0,)), ((), ())),
                           preferred_element_type=jnp.float32)


def _p4(a):
    return a.reshape(a.shape[0] // 4, 4 * a.shape[1])


def _bd(w):
    return jnp.kron(jnp.eye(4, dtype=w.dtype), w)


def _t4(b):
    return jnp.tile(b.reshape(1, -1), (1, 4))


def _quarter_onehots(bb):
    iota = lax.broadcasted_iota(jnp.int32, (1, _G), 1)
    return [(bb[:, j:j + 1] == iota).astype(jnp.float32) for j in range(4)]


def _node_proj_body(x_ref, wa_ref, wb_ref, wc_ref, a_ref, b_ref, c_ref):
    x = x_ref[...]
    a_ref[...] = _dot(x, wa_ref[...])
    b_ref[...] = _dot(x, wb_ref[...])
    c_ref[...] = _dot(x, wc_ref[...])


def _node_proj(x, w1a, w1b, wv1a):
    d = x.shape[1]
    tn = 1000
    return pl.pallas_call(
        _node_proj_body,
        grid=(_N // tn,),
        in_specs=[
            pl.BlockSpec((tn, d), lambda i: (i, 0)),
            pl.BlockSpec((d, _F), lambda i: (0, 0)),
            pl.BlockSpec((d, _F), lambda i: (0, 0)),
            pl.BlockSpec((d, _F), lambda i: (0, 0)),
        ],
        out_specs=[pl.BlockSpec((tn, _F), lambda i: (i, 0))] * 3,
        out_shape=[jax.ShapeDtypeStruct((_N, _F), jnp.float32)] * 3,
    )(x, w1a, w1b, wv1a)


def _edge_body(g_ref, ea_ref, bb_ref, u_ref, w1c_ref, w1d_ref, b1_ref,
               w2_ref, b2_ref, w3_ref, b3_ref, e_ref, ue_ref, gc_ref):
    i = pl.program_id(0)
    ohs = _quarter_onehots(bb_ref[0])
    u1 = _dot(u_ref[...], w1d_ref[...])
    uterm = jnp.concatenate([_dot(oh, u1) for oh in ohs], axis=1)
    h = g_ref[...] + _dot(ea_ref[...], w1c_ref[...]) + uterm + b1_ref[...]
    h = _sp(h)
    h = _sp(_dot(h, w2_ref[...]) + b2_ref[...])
    h = _sp(_dot(h, w3_ref[...]) + b3_ref[...])
    e_ref[...] = h
    ones = jnp.ones((h.shape[0], 1), jnp.float32)
    part = sum(_dg0(ohs[j], h[:, 32 * j:32 * j + 32]) for j in range(4))
    cnt = sum(_dg0(ohs[j], ones) for j in range(4))

    @pl.when(i == 0)
    def _():
        ue_ref[...] = part
        gc_ref[...] = cnt

    @pl.when(i > 0)
    def _():
        ue_ref[...] = ue_ref[...] + part
        gc_ref[...] = gc_ref[...] + cnt


def _edge_mlp(g4, ea4, bb4, u, w1c_blk, w1d, b1t, w2_blk, b2t, w3_blk, b3t):
    ein4 = ea4.shape[1]
    ud = u.shape[1]
    return pl.pallas_call(
        _edge_body,
        grid=(_NTE4,),
        in_specs=[
            pl.BlockSpec((_TE4, 128), lambda i: (i, 0)),
            pl.BlockSpec((_TE4, ein4), lambda i: (i, 0)),
            pl.BlockSpec((1, _TE4, 4), lambda i: (i, 0, 0)),
            pl.BlockSpec((_G, ud), lambda i: (0, 0)),
            pl.BlockSpec((ein4, 128), lambda i: (0, 0)),
            pl.BlockSpec((ud, _F), lambda i: (0, 0)),
            pl.BlockSpec((1, 128), lambda i: (0, 0)),
            pl.BlockSpec((128, 128), lambda i: (0, 0)),
            pl.BlockSpec((1, 128), lambda i: (0, 0)),
            pl.BlockSpec((128, 128), lambda i: (0, 0)),
            pl.BlockSpec((1, 128), lambda i: (0, 0)),
        ],
        out_specs=[
            pl.BlockSpec((_TE4, 128), lambda i: (i, 0)),
            pl.BlockSpec((_G, _F), lambda i: (0, 0)),
            pl.BlockSpec((_G, 1), lambda i: (0, 0)),
        ],
        out_shape=[
            jax.ShapeDtypeStruct((_E // 4, 128), jnp.float32),
            jax.ShapeDtypeStruct((_G, _F), jnp.float32),
            jax.ShapeDtypeStruct((_G, 1), jnp.float32),
        ],
    )(g4, ea4, bb4, u, w1c_blk, w1d, b1t, w2_blk, b2t, w3_blk, b3t)


def _make_node_body(n_extra):
    def body(*refs):
        (vx_ref, es_ref, cn_ref, b_ref, u_ref, wv1b_ref, wv1c_ref, bv1_ref,
         wv2_ref, bv2_ref, wv3_ref, bv3_ref) = refs[:12]
        ew_refs = refs[12:12 + n_extra]
        v_ref, uv_ref, nc_ref = refs[12 + n_extra:15 + n_extra]
        ex_refs = refs[15 + n_extra:]
        i = pl.program_id(0)
        es = es_ref[0] + es_ref[1]
        cn = cn_ref[0] + cn_ref[1]
        div = jnp.concatenate(
            [jnp.broadcast_to(cn[:, 16 * j:16 * j + 1], (_TN4, 32))
             for j in range(4)], axis=1)
        emean = es / jnp.maximum(div, 1.0)
        ohs = _quarter_onehots(b_ref[0])
        u1 = _dot(u_ref[...], wv1c_ref[...])
        uterm = jnp.concatenate([_dot(oh, u1) for oh in ohs], axis=1)
        h = vx_ref[...] + _dot(emean, wv1b_ref[...]) + uterm + bv1_ref[...]
        h = _sp(h)
        h = _sp(_dot(h, wv2_ref[...]) + bv2_ref[...])
        v = _sp(_dot(h, wv3_ref[...]) + bv3_ref[...])
        v_ref[...] = v
        for ew, ex in zip(ew_refs, ex_refs):
            ex[...] = _dot(v, ew[...])
        ones = jnp.ones((v.shape[0], 1), jnp.float32)
        part = sum(_dg0(ohs[j], v[:, 32 * j:32 * j + 32]) for j in range(4))
        cnt = sum(_dg0(ohs[j], ones) for j in range(4))

        @pl.when(i == 0)
        def _():
            uv_ref[...] = part
            nc_ref[...] = cnt

        @pl.when(i > 0)
        def _():
            uv_ref[...] = uv_ref[...] + part
            nc_ref[...] = nc_ref[...] + cnt

    return body


def _node_mlp(vx4, es4, cn4, nb4, u, wv1b_blk, wv1c, bv1t, wv2_blk, bv2t,
              wv3_blk, bv3t, extras_w):
    ud = u.shape[1]
    full = lambda r, c: pl.BlockSpec((r, c), lambda i: (0, 0))
    in_specs = [
        pl.BlockSpec((_TN4, 128), lambda i: (i, 0)),
        pl.BlockSpec((2, _TN4, 128), lambda i: (0, i, 0)),
        pl.BlockSpec((2, _TN4, 64), lambda i: (0, i, 0)),
        pl.BlockSpec((1, _TN4, 4), lambda i: (i, 0, 0)),
        full(_G, ud), full(128, 128), full(ud, _F), full(1, 128),
        full(128, 128), full(1, 128), full(128, 128), full(1, 128),
    ] + [full(128, w.shape[1]) for w in extras_w]
    out_specs = [
        pl.BlockSpec((_TN4, 128), lambda i: (i, 0)),
        pl.BlockSpec((_G, _F), lambda i: (0, 0)),
        pl.BlockSpec((_G, 1), lambda i: (0, 0)),
    ] + [pl.BlockSpec((_TN4, w.shape[1]), lambda i: (i, 0))
         for w in extras_w]
    out_shape = [
        jax.ShapeDtypeStruct((_N // 4, 128), jnp.float32),
        jax.ShapeDtypeStruct((_G, _F), jnp.float32),
        jax.ShapeDtypeStruct((_G, 1), jnp.float32),
    ] + [jax.ShapeDtypeStruct((_N // 4, w.shape[1]), jnp.float32)
         for w in extras_w]
    return pl.pallas_call(
        _make_node_body(len(extras_w)),
        grid=(_NTN4,),
        in_specs=in_specs,
        out_specs=out_specs,
        out_shape=out_shape,
    )(vx4, es4, cn4, nb4, u, wv1b_blk, wv1c, bv1t, wv2_blk, bv2t, wv3_blk,
      bv3t, *extras_w)


def _u_body(ue_ref, gc_ref, uv_ref, nc_ref, u_ref, wue_ref, wuv_ref, wuu_ref,
            bu1_ref, wu2_ref, bu2_ref, wu3_ref, bu3_ref, out_ref):
    ue = ue_ref[...] / jnp.maximum(gc_ref[...], 1.0)
    uv = uv_ref[...] / jnp.maximum(nc_ref[...], 1.0)
    h = _sp(_dot(ue, wue_ref[...]) + _dot(uv, wuv_ref[...])
            + _dot(u_ref[...], wuu_ref[...]) + bu1_ref[...])
    h = _sp(_dot(h, wu2_ref[...]) + bu2_ref[...])
    out_ref[...] = _sp(_dot(h, wu3_ref[...]) + bu3_ref[...])


def _u_mlp(ue_sum, gcnt, uv_sum, ncnt, u, wue, wuv, wuu, bu1, wu2, bu2,
           wu3, bu3):
    ud = u.shape[1]
    full = lambda shape: pl.BlockSpec(shape, lambda: (0,) * len(shape))
    return pl.pallas_call(
        _u_body,
        in_specs=[
            full((_G, _F)), full((_G, 1)), full((_G, _F)), full((_G, 1)),
            full((_G, ud)), full((_F, _F)), full((_F, _F)), full((ud, _F)),
            full((1, _F)), full((_F, _F)), full((1, _F)), full((_F, _F)),
            full((1, _F)),
        ],
        out_specs=full((_G, _F)),
        out_shape=jax.ShapeDtypeStruct((_G, _F), jnp.float32),
    )(ue_sum, gcnt, uv_sum, ncnt, u, wue, wuv, wuu, bu1, wu2, bu2, wu3, bu3)


def _head_ij_body(r_ref, e_ref, bb_ref, u_ref, wc_ref, wd_ref, b1_ref,
                  w2_ref, b2_ref, w3_ref, b3_ref, out_ref):
    ohs = _quarter_onehots(bb_ref[0])
    u1 = _dot(u_ref[...], wd_ref[...])
    uterm = jnp.concatenate([_dot(oh, u1) for oh in ohs], axis=1)
    r4 = r_ref[...]
    rterm = jnp.concatenate([r4[:, 16 * j:16 * j + 4] for j in range(4)],
                            axis=1)
    h = rterm + _dot(e_ref[...], wc_ref[...]) + uterm + b1_ref[...]
    h = _sp(h)
    h = _sp(_dot(h, w2_ref[...]) + b2_ref[...])
    out_ref[...] = _sp(_dot(h, w3_ref[...]) + b3_ref[...])


def _head_ij(r4, e4, bb4, u, wc_blk, wd, b1t, w2_blk, b2t, w3_blk, b3t):
    return pl.pallas_call(
        _head_ij_body,
        grid=(_NTE4,),
        in_specs=[
            pl.BlockSpec((_TE4, 64), lambda i: (i, 0)),
            pl.BlockSpec((_TE4, 128), lambda i: (i, 0)),
            pl.BlockSpec((1, _TE4, 4), lambda i: (i, 0, 0)),
            pl.BlockSpec((_G, _F), lambda i: (0, 0)),
            pl.BlockSpec((128, 16), lambda i: (0, 0)),
            pl.BlockSpec((_F, 4), lambda i: (0, 0)),
            pl.BlockSpec((1, 16), lambda i: (0, 0)),
            pl.BlockSpec((16, 16), lambda i: (0, 0)),
            pl.BlockSpec((1, 16), lambda i: (0, 0)),
            pl.BlockSpec((16, 16), lambda i: (0, 0)),
            pl.BlockSpec((1, 16), lambda i: (0, 0)),
        ],
        out_specs=pl.BlockSpec((_TE4, 16), lambda i: (i, 0)),
        out_shape=jax.ShapeDtypeStruct((_E // 4, 16), jnp.float32),
    )(r4, e4, bb4, u, wc_blk, wd, b1t, w2_blk, b2t, w3_blk, b3t)


def _head_ii_body(v_ref, b_ref, u_ref, wa_ref, wb_ref, b1_ref,
                  w2_ref, b2_ref, w3_ref, b3_ref, out_ref):
    ohs = _quarter_onehots(b_ref[0])
    u1 = _dot(u_ref[...], wb_ref[...])
    uterm = jnp.concatenate([_dot(oh, u1) for oh in ohs], axis=1)
    h = _sp(_dot(v_ref[...], wa_ref[...]) + uterm + b1_ref[...])
    h = _sp(_dot(h, w2_ref[...]) + b2_ref[...])
    out_ref[...] = _sp(_dot(h, w3_ref[...]) + b3_ref[...])


def _head_ii(v4, nb4, u, wa_blk, wb, b1t, w2_blk, b2t, w3_blk, b3t):
    return pl.pallas_call(
        _head_ii_body,
        grid=(_NTN4,),
        in_specs=[
            pl.BlockSpec((_TN4, 128), lambda i: (i, 0)),
            pl.BlockSpec((1, _TN4, 4), lambda i: (i, 0, 0)),
            pl.BlockSpec((_G, _F), lambda i: (0, 0)),
            pl.BlockSpec((128, 16), lambda i: (0, 0)),
            pl.BlockSpec((_F, 4), lambda i: (0, 0)),
            pl.BlockSpec((1, 16), lambda i: (0, 0)),
            pl.BlockSpec((16, 16), lambda i: (0, 0)),
            pl.BlockSpec((1, 16), lambda i: (0, 0)),
            pl.BlockSpec((16, 16), lambda i: (0, 0)),
            pl.BlockSpec((1, 16), lambda i: (0, 0)),
        ],
        out_specs=pl.BlockSpec((_TN4, 16), lambda i: (i, 0)),
        out_shape=jax.ShapeDtypeStruct((_N // 4, 16), jnp.float32),
    )(v4, nb4, u, wa_blk, wb, b1t, w2_blk, b2t, w3_blk, b3t)


def kernel(x, edge_index, edge_attr, state, batch, bond_batch, params):
    src = edge_index[0]
    dst = edge_index[1]
    bb4 = bond_batch.reshape(_NTE4, _TE4, 4)
    nb4 = batch.reshape(_NTN4, _TN4, 4)

    names = ("embed", "core0", "core1")
    (Wj1, bj1), (Wj2, bj2), (Wj3, bj3) = params["head_ij"]
    wja_pad = jnp.pad(Wj1[:_F], ((0, 0), (0, 12)))
    wjb_pad = jnp.pad(Wj1[_F:2 * _F], ((0, 0), (0, 12)))

    cnt4 = None
    u = state
    ein4 = _p4(edge_attr)
    for b, name in enumerate(names):
        bp = params[name]
        (W1, b1), (W2, b2), (W3, b3) = bp["phi_e"]
        n_in = 128 if b == 0 else _F
        e_in = ein4.shape[1] // 4
        w1a = W1[:n_in]
        w1b = W1[n_in:2 * n_in]
        w1c = W1[2 * n_in:2 * n_in + e_in]
        w1d = W1[2 * n_in + e_in:]
        (Wv1, bv1), (Wv2, bv2), (Wv3, bv3) = bp["phi_v"]
        wv1a = Wv1[:n_in]
        wv1b = Wv1[n_in:n_in + _F]
        wv1c = Wv1[n_in + _F:]

        if b == 0:
            A, B, VX = _node_proj(x, w1a, w1b, wv1a)
            vx4 = _p4(VX)

        Gm = _gather_rows_add(A, B, src, dst, _F)
        e4, ue_sum, gcnt = _edge_mlp(
            _p4(Gm), ein4, bb4, u, _bd(w1c), w1d, _t4(b1), _bd(W2),
            _t4(b2), _bd(W3), _t4(b3))
        if cnt4 is None:
            es_part, cnt_part = _scatter_count_partials(
                e4.reshape(_E, _F), dst)
            cnt4 = cnt_part.reshape(2, _N // 4, 64)
        else:
            es_part = _scatter_partials(e4.reshape(_E, _F), dst)
        es4 = es_part.reshape(2, _N // 4, 128)

        if b < 2:
            W1n = params[names[b + 1]]["phi_e"][0][0]
            Wv1n = params[names[b + 1]]["phi_v"][0][0]
            extras_w = [_bd(W1n[:_F]), _bd(W1n[_F:2 * _F]), _bd(Wv1n[:_F])]
        else:
            extras_w = [_bd(wja_pad), _bd(wjb_pad)]
        outs = _node_mlp(
            vx4, es4, cnt4, nb4, u, _bd(wv1b), wv1c, _t4(bv1), _bd(Wv2),
            _t4(bv2), _bd(Wv3), _t4(bv3), extras_w)
        v4, uv_sum, ncnt = outs[:3]
        if b < 2:
            A = outs[3].reshape(_N, _F)
            B = outs[4].reshape(_N, _F)
            vx4 = outs[5]
        else:
            P = outs[3].reshape(_N, 16)
            Q = outs[4].reshape(_N, 16)

        (Wu1, bu1), (Wu2, bu2), (Wu3, bu3) = bp["phi_u"]
        u = _u_mlp(ue_sum, gcnt, uv_sum, ncnt, u, Wu1[:_F], Wu1[_F:2 * _F],
                   Wu1[2 * _F:], bu1.reshape(1, -1), Wu2, bu2.reshape(1, -1),
                   Wu3, bu3.reshape(1, -1))
        ein4 = e4

    (Wi1, bi1), (Wi2, bi2), (Wi3, bi3) = params["head_ii"]
    ham_ii4 = _head_ii(v4, nb4, u, _bd(Wi1[:_F]), Wi1[_F:], _t4(bi1),
                       _bd(Wi2), _t4(bi2), _bd(Wi3), _t4(bi3))

    R = _gather_rows_add(P, Q, src, dst, 16)
    ham_ij4 = _head_ij(R.reshape(_E // 4, 64), ein4, bb4, u,
                       _bd(Wj1[2 * _F:3 * _F]), Wj1[3 * _F:], _t4(bj1),
                       _bd(Wj2), _t4(bj2), _bd(Wj3), _t4(bj3))

    return (ham_ii4.reshape(_N, 4), ham_ij4.reshape(_E, 4), edge_index.T)


# speedup vs baseline: 7.3501x; 1.0450x over previous
import functools

import jax
import jax.numpy as jnp
from jax import lax
from jax.experimental import pallas as pl
from jax.experimental.pallas import tpu as pltpu
from jax.experimental.pallas import tpu_sc as plsc

_N = 10000
_E = 320000
_G = 64
_F = 32

_TE4 = 1000
_NTE4 = _E // 4 // _TE4
_TN4 = 2500
_NTN4 = _N // 4 // _TN4

_NW = 32
_ZR = _N // 16


def _sp(x):
    return jnp.maximum(x, 0.0) + jnp.log(1.0 + jnp.exp(-jnp.abs(x)))


def _mesh():
    return plsc.VectorSubcoreMesh(core_axis_name="c", subcore_axis_name="s")


_SC_PARAMS = pltpu.CompilerParams(use_tc_tiling_on_sc=False)


_CM = 400
_NM = _E // _NW // _CM
_GS = 80


@functools.lru_cache(maxsize=None)
def _gather_add_fn(width):
    nsl = width // 16

    def body(a_hbm, b_hbm, src_hbm, dst_hbm, out_hbm,
             si0, si1, di0, di1, ra0, ra1, rb0, rb1,
             sI0, sI1, sG0, sG1, sW0, sW1):
        si = (si0, si1)
        di = (di0, di1)
        ra = (ra0, ra1)
        rb = (rb0, rb1)
        sI = (sI0, sI1)
        sG = (sG0, sG1)
        sW = (sW0, sW1)
        w = lax.axis_index("s") * 2 + lax.axis_index("c")
        base = w * (_E // _NW)

        def issue_gathers(q):
            for j in range(_CM // _GS):
                sl = pl.ds(j * _GS, _GS)
                pltpu.async_copy(a_hbm.at[si[q].at[sl]], ra[q].at[sl], sG[q])
                pltpu.async_copy(b_hbm.at[di[q].at[sl]], rb[q].at[sl], sG[q])

        def wait_gathers(q):
            for j in range(_CM // _GS):
                sl = pl.ds(j * _GS, _GS)
                pltpu.make_async_copy(
                    a_hbm.at[si[q].at[sl]], ra[q].at[sl], sG[q]).wait()
                pltpu.make_async_copy(
                    b_hbm.at[di[q].at[sl]], rb[q].at[sl], sG[q]).wait()

        i0a = pltpu.async_copy(src_hbm.at[pl.ds(base, _CM)], si[0], sI[0])
        i0b = pltpu.async_copy(dst_hbm.at[pl.ds(base, _CM)], di[0], sI[0])
        i0a.wait()
        i0b.wait()
        issue_gathers(0)
        pltpu.async_copy(src_hbm.at[pl.ds(base + _CM, _CM)], si[1], sI[1])
        pltpu.async_copy(dst_hbm.at[pl.ds(base + _CM, _CM)], di[1], sI[1])

        def outer(mm, carry):
            for p in range(2):
                m = mm * 2 + p

                @pl.when(m < _NM)
                def _():
                    off = base + m * _CM
                    wait_gathers(p)

                    @pl.when(m + 2 < _NM)
                    def _():
                        off2 = off + 2 * _CM
                        pltpu.async_copy(
                            src_hbm.at[pl.ds(off2, _CM)], si[p], sI[p])
                        pltpu.async_copy(
                            dst_hbm.at[pl.ds(off2, _CM)], di[p], sI[p])

                    @pl.when(m + 1 < _NM)
                    def _():
                        pltpu.make_async_copy(
                            src_hbm.at[pl.ds(0, _CM)], si[1 - p],
                            sI[1 - p]).wait()
                        pltpu.make_async_copy(
                            dst_hbm.at[pl.ds(0, _CM)], di[1 - p],
                            sI[1 - p]).wait()

                        @pl.when(m >= 1)
                        def _():
                            pltpu.make_async_copy(
                                ra[1 - p], out_hbm.at[pl.ds(0, _CM)],
                                sW[1 - p]).wait()

                        issue_gathers(1 - p)

                    def add8(it, c2):
                        r0 = it * 8
                        for rr in range(8):
                            for jj in range(nsl):
                                s2 = pl.ds(jj * 16, 16)
                                ra[p][r0 + rr, s2] = (
                                    ra[p][r0 + rr, s2] + rb[p][r0 + rr, s2])
                        return c2

                    lax.fori_loop(0, _CM // 8, add8, 0)
                    pltpu.async_copy(ra[p], out_hbm.at[pl.ds(off, _CM)], sW[p])

            return carry

        lax.fori_loop(0, (_NM + 1) // 2, outer, 0)
        pltpu.make_async_copy(ra[1], out_hbm.at[pl.ds(0, _CM)], sW[1]).wait()
        pltpu.make_async_copy(ra[0], out_hbm.at[pl.ds(0, _CM)], sW[0]).wait()

    return pl.kernel(
        body,
        mesh=_mesh(),
        compiler_params=_SC_PARAMS,
        out_type=jax.ShapeDtypeStruct((_E, width), jnp.float32),
        scratch_types=[
            pltpu.VMEM((_CM,), jnp.int32),
            pltpu.VMEM((_CM,), jnp.int32),
            pltpu.VMEM((_CM,), jnp.int32),
            pltpu.VMEM((_CM,), jnp.int32),
            pltpu.VMEM((_CM, width), jnp.float32),
            pltpu.VMEM((_CM, width), jnp.float32),
            pltpu.VMEM((_CM, width), jnp.float32),
            pltpu.VMEM((_CM, width), jnp.float32),
            pltpu.SemaphoreType.DMA,
            pltpu.SemaphoreType.DMA,
            pltpu.SemaphoreType.DMA,
            pltpu.SemaphoreType.DMA,
            pltpu.SemaphoreType.DMA,
            pltpu.SemaphoreType.DMA,
        ],
    )


@functools.lru_cache(maxsize=None)
def _scatter_fn(with_count):
    width = _F

    def body(*args):
        if with_count:
            (e_hbm, dst_hbm, out_hbm, cout_hbm,
             si0, si1, eb0, eb1, ones, zb, zb2, acc, cacc,
             sI0, sI1, sE0, sE1, sO) = args
        else:
            (e_hbm, dst_hbm, out_hbm,
             si0, si1, eb0, eb1, zb, acc,
             sI0, sI1, sE0, sE1, sO) = args
        si = (si0, si1)
        eb = (eb0, eb1)
        sI = (sI0, sI1)
        sE = (sE0, sE1)
        cid = lax.axis_index("c")
        sid = lax.axis_index("s")
        w = sid * 2 + cid
        base = w * (_E // _NW)

        def zrow(i, carry):
            for jj in range(width // 16):
                zb[i, pl.ds(jj * 16, 16)] = jnp.zeros((16,), jnp.float32)
            if with_count:
                zb2[i, pl.ds(0, 16)] = jnp.zeros((16,), jnp.float32)
            return carry

        lax.fori_loop(0, _ZR, zrow, 0)
        if with_count:
            def orow(i, carry):
                ones[i, pl.ds(0, 16)] = jnp.ones((16,), jnp.float32)
                return carry

            lax.fori_loop(0, _CM, orow, 0)
        pltpu.async_copy(dst_hbm.at[pl.ds(base, _CM)], si[0], sI[0])
        pltpu.async_copy(e_hbm.at[pl.ds(base, _CM)], eb[0], sE[0])
        pltpu.sync_copy(zb, acc.at[pl.ds(sid * _ZR, _ZR)])
        if with_count:
            pltpu.sync_copy(zb2, cacc.at[pl.ds(sid * _ZR, _ZR)])
        plsc.subcore_barrier()

        def outer(mm, carry):
            for p in range(2):
                m = mm * 2 + p

                @pl.when(m < _NM)
                def _():
                    off = base + m * _CM
                    pltpu.make_async_copy(
                        dst_hbm.at[pl.ds(0, _CM)], si[p], sI[p]).wait()
                    pltpu.make_async_copy(
                        e_hbm.at[pl.ds(0, _CM)], eb[p], sE[p]).wait()

                    @pl.when(m + 1 < _NM)
                    def _():
                        off2 = off + _CM
                        pltpu.async_copy(
                            dst_hbm.at[pl.ds(off2, _CM)], si[1 - p], sI[1 - p])
                        pltpu.async_copy(
                            e_hbm.at[pl.ds(off2, _CM)], eb[1 - p], sE[1 - p])

                    pltpu.sync_copy(eb[p], acc.at[si[p]], add=True)
                    if with_count:
                        pltpu.sync_copy(ones, cacc.at[si[p]], add=True)

            return carry

        lax.fori_loop(0, (_NM + 1) // 2, outer, 0)
        plsc.subcore_barrier()
        rows = pl.ds(sid * _ZR, _ZR)
        cps = [pltpu.async_copy(acc.at[rows], out_hbm.at[cid, rows], sO)]
        if with_count:
            cps.append(pltpu.async_copy(cacc.at[rows], cout_hbm.at[cid, rows],
                                        sO))
        for cp in cps:
            cp.wait()

    out_type = [jax.ShapeDtypeStruct((2, _N, width), jnp.float32)]
    scratch = [
        pltpu.VMEM((_CM,), jnp.int32),
        pltpu.VMEM((_CM,), jnp.int32),
        pltpu.VMEM((_CM, width), jnp.float32),
        pltpu.VMEM((_CM, width), jnp.float32),
    ]
    if with_count:
        out_type.append(jax.ShapeDtypeStruct((2, _N, 16), jnp.float32))
        scratch.append(pltpu.VMEM((_CM, 16), jnp.float32))
    scratch.append(pltpu.VMEM((_ZR, width), jnp.float32))
    if with_count:
        scratch.append(pltpu.VMEM((_ZR, 16), jnp.float32))
    scratch.append(pltpu.VMEM_SHARED((_N, width), jnp.float32))
    if with_count:
        scratch.append(pltpu.VMEM_SHARED((_N, 16), jnp.float32))
    scratch += [pltpu.SemaphoreType.DMA] * 5
    return pl.kernel(
        body,
        mesh=_mesh(),
        compiler_params=_SC_PARAMS,
        out_type=out_type if with_count else out_type[0],
        scratch_types=scratch,
    )


def _gather_rows_add(a, b, src, dst, width):
    return _gather_add_fn(width)(a, b, src, dst)


def _scatter_partials(e, dst):
    return _scatter_fn(False)(e, dst)


def _scatter_count_partials(e, dst):
    return _scatter_fn(True)(e, dst)


def _dot(a, b):
    return jnp.dot(a, b, preferred_element_type=jnp.float32)


def _dg0(a, b):
    return lax.dot_general(a, b, (((0,), (0,)), ((), ())),
                           preferred_element_type=jnp.float32)


def _p4(a):
    return a.reshape(a.shape[0] // 4, 4 * a.shape[1])


def _bd(w):
    return jnp.kron(jnp.eye(4, dtype=w.dtype), w)


def _t4(b):
    return jnp.tile(b.reshape(1, -1), (1, 4))


def _quarter_onehots(bb):
    iota = lax.broadcasted_iota(jnp.int32, (1, _G), 1)
    return [(bb[:, j:j + 1] == iota).astype(jnp.float32) for j in range(4)]


def _node_proj_body(x_ref, wa_ref, wb_ref, wc_ref, a_ref, b_ref, c_ref):
    x = x_ref[...]
    a_ref[...] = _dot(x, wa_ref[...])
    b_ref[...] = _dot(x, wb_ref[...])
    c_ref[...] = _dot(x, wc_ref[...])


def _node_proj(x, w1a, w1b, wv1a):
    d = x.shape[1]
    tn = 1000
    return pl.pallas_call(
        _node_proj_body,
        grid=(_N // tn,),
        in_specs=[
            pl.BlockSpec((tn, d), lambda i: (i, 0)),
            pl.BlockSpec((d, _F), lambda i: (0, 0)),
            pl.BlockSpec((d, _F), lambda i: (0, 0)),
            pl.BlockSpec((d, _F), lambda i: (0, 0)),
        ],
        out_specs=[pl.BlockSpec((tn, _F), lambda i: (i, 0))] * 3,
        out_shape=[jax.ShapeDtypeStruct((_N, _F), jnp.float32)] * 3,
    )(x, w1a, w1b, wv1a)


def _edge_body(g_ref, ea_ref, bb_ref, u_ref, w1c_ref, w1d_ref, b1_ref,
               w2_ref, b2_ref, w3_ref, b3_ref, e_ref, ue_ref, gc_ref):
    i = pl.program_id(0)
    ohs = _quarter_onehots(bb_ref[0])
    u1 = _dot(u_ref[...], w1d_ref[...])
    uterm = jnp.concatenate([_dot(oh, u1) for oh in ohs], axis=1)
    h = g_ref[...] + _dot(ea_ref[...], w1c_ref[...]) + uterm + b1_ref[...]
    h = _sp(h)
    h = _sp(_dot(h, w2_ref[...]) + b2_ref[...])
    h = _sp(_dot(h, w3_ref[...]) + b3_ref[...])
    e_ref[...] = h
    ones = jnp.ones((h.shape[0], 1), jnp.float32)
    part = sum(_dg0(ohs[j], h[:, 32 * j:32 * j + 32]) for j in range(4))
    cnt = sum(_dg0(ohs[j], ones) for j in range(4))

    @pl.when(i == 0)
    def _():
        ue_ref[...] = part
        gc_ref[...] = cnt

    @pl.when(i > 0)
    def _():
        ue_ref[...] = ue_ref[...] + part
        gc_ref[...] = gc_ref[...] + cnt


def _edge_mlp(g4, ea4, bb4, u, w1c_blk, w1d, b1t, w2_blk, b2t, w3_blk, b3t):
    ein4 = ea4.shape[1]
    ud = u.shape[1]
    return pl.pallas_call(
        _edge_body,
        grid=(_NTE4,),
        in_specs=[
            pl.BlockSpec((_TE4, 128), lambda i: (i, 0)),
            pl.BlockSpec((_TE4, ein4), lambda i: (i, 0)),
            pl.BlockSpec((1, _TE4, 4), lambda i: (i, 0, 0)),
            pl.BlockSpec((_G, ud), lambda i: (0, 0)),
            pl.BlockSpec((ein4, 128), lambda i: (0, 0)),
            pl.BlockSpec((ud, _F), lambda i: (0, 0)),
            pl.BlockSpec((1, 128), lambda i: (0, 0)),
            pl.BlockSpec((128, 128), lambda i: (0, 0)),
            pl.BlockSpec((1, 128), lambda i: (0, 0)),
            pl.BlockSpec((128, 128), lambda i: (0, 0)),
            pl.BlockSpec((1, 128), lambda i: (0, 0)),
        ],
        out_specs=[
            pl.BlockSpec((_TE4, 128), lambda i: (i, 0)),
            pl.BlockSpec((_G, _F), lambda i: (0, 0)),
            pl.BlockSpec((_G, 1), lambda i: (0, 0)),
        ],
        out_shape=[
            jax.ShapeDtypeStruct((_E // 4, 128), jnp.float32),
            jax.ShapeDtypeStruct((_G, _F), jnp.float32),
            jax.ShapeDtypeStruct((_G, 1), jnp.float32),
        ],
    )(g4, ea4, bb4, u, w1c_blk, w1d, b1t, w2_blk, b2t, w3_blk, b3t)


def _make_node_body(n_extra, with_head):
    def body(*refs):
        (vx_ref, es_ref, cn_ref, b_ref, u_ref, wv1b_ref, wv1c_ref, bv1_ref,
         wv2_ref, bv2_ref, wv3_ref, bv3_ref, ue_ref, gc_ref, wue_ref,
         wuv_ref, wuu_ref, bu1_ref, wu2_ref, bu2_ref, wu3_ref,
         bu3_ref) = refs[:22]
        k = 22
        ew_refs = refs[k:k + n_extra]
        k += n_extra
        if with_head:
            (wia_ref, wib_ref, bi1_ref, wi2_ref, bi2_ref, wi3_ref,
             bi3_ref) = refs[k:k + 7]
            k += 7
        v_ref = refs[k]
        unew_ref = refs[k + 1]
        ex_refs = refs[k + 2:k + 2 + n_extra]
        if with_head:
            hii_ref = refs[k + 2 + n_extra]
        es = es_ref[0] + es_ref[1]
        cn = cn_ref[0] + cn_ref[1]
        div = jnp.concatenate(
            [jnp.broadcast_to(cn[:, 16 * j:16 * j + 1], (_TN4, 32))
             for j in range(4)], axis=1)
        emean = es / jnp.maximum(div, 1.0)
        ohs = _quarter_onehots(b_ref[0])
        u = u_ref[...]
        u1 = _dot(u, wv1c_ref[...])
        uterm = jnp.concatenate([_dot(oh, u1) for oh in ohs], axis=1)
        h = vx_ref[...] + _dot(emean, wv1b_ref[...]) + uterm + bv1_ref[...]
        h = _sp(h)
        h = _sp(_dot(h, wv2_ref[...]) + bv2_ref[...])
        v = _sp(_dot(h, wv3_ref[...]) + bv3_ref[...])
        v_ref[...] = v
        for ew, ex in zip(ew_refs, ex_refs):
            ex[...] = _dot(v, ew[...])
        ones = jnp.ones((v.shape[0], 1), jnp.float32)
        uv_sum = sum(_dg0(ohs[j], v[:, 32 * j:32 * j + 32]) for j in range(4))
        ncnt = sum(_dg0(ohs[j], ones) for j in range(4))
        ue = ue_ref[...] / jnp.maximum(gc_ref[...], 1.0)
        uv = uv_sum / jnp.maximum(ncnt, 1.0)
        hu = _sp(_dot(ue, wue_ref[...]) + _dot(uv, wuv_ref[...])
                 + _dot(u, wuu_ref[...]) + bu1_ref[...])
        hu = _sp(_dot(hu, wu2_ref[...]) + bu2_ref[...])
        u_new = _sp(_dot(hu, wu3_ref[...]) + bu3_ref[...])
        unew_ref[...] = u_new
        if with_head:
            u1h = _dot(u_new, wib_ref[...])
            uth = jnp.concatenate([_dot(oh, u1h) for oh in ohs], axis=1)
            hh = _sp(_dot(v, wia_ref[...]) + uth + bi1_ref[...])
            hh = _sp(_dot(hh, wi2_ref[...]) + bi2_ref[...])
            hii_ref[...] = _sp(_dot(hh, wi3_ref[...]) + bi3_ref[...])

    return body


def _node_mlp(vx4, es4, cn4, nb4, u, wv1b_blk, wv1c, bv1t, wv2_blk, bv2t,
              wv3_blk, bv3t, ue_sum, gcnt, u_ws, extras_w, head_ws):
    ud = u.shape[1]
    full = lambda r, c: pl.BlockSpec((r, c), lambda i: (0, 0))
    in_specs = [
        pl.BlockSpec((_TN4, 128), lambda i: (i, 0)),
        pl.BlockSpec((2, _TN4, 128), lambda i: (0, i, 0)),
        pl.BlockSpec((2, _TN4, 64), lambda i: (0, i, 0)),
        pl.BlockSpec((1, _TN4, 4), lambda i: (i, 0, 0)),
        full(_G, ud), full(128, 128), full(ud, _F), full(1, 128),
        full(128, 128), full(1, 128), full(128, 128), full(1, 128),
        full(_G, _F), full(_G, 1),
    ] + [full(*w.shape) for w in u_ws] \
      + [full(128, w.shape[1]) for w in extras_w] \
      + [full(*w.shape) for w in head_ws]
    out_specs = [
        pl.BlockSpec((_TN4, 128), lambda i: (i, 0)),
        pl.BlockSpec((_G, _F), lambda i: (0, 0)),
    ] + [pl.BlockSpec((_TN4, w.shape[1]), lambda i: (i, 0))
         for w in extras_w]
    out_shape = [
        jax.ShapeDtypeStruct((_N // 4, 128), jnp.float32),
        jax.ShapeDtypeStruct((_G, _F), jnp.float32),
    ] + [jax.ShapeDtypeStruct((_N // 4, w.shape[1]), jnp.float32)
         for w in extras_w]
    if head_ws:
        out_specs.append(pl.BlockSpec((_TN4, 16), lambda i: (i, 0)))
        out_shape.append(jax.ShapeDtypeStruct((_N // 4, 16), jnp.float32))
    return pl.pallas_call(
        _make_node_body(len(extras_w), bool(head_ws)),
        grid=(_NTN4,),
        in_specs=in_specs,
        out_specs=out_specs,
        out_shape=out_shape,
    )(vx4, es4, cn4, nb4, u, wv1b_blk, wv1c, bv1t, wv2_blk, bv2t, wv3_blk,
      bv3t, ue_sum, gcnt, *u_ws, *extras_w, *head_ws)


def _u_body(ue_ref, gc_ref, uv_ref, nc_ref, u_ref, wue_ref, wuv_ref, wuu_ref,
            bu1_ref, wu2_ref, bu2_ref, wu3_ref, bu3_ref, out_ref):
    ue = ue_ref[...] / jnp.maximum(gc_ref[...], 1.0)
    uv = uv_ref[...] / jnp.maximum(nc_ref[...], 1.0)
    h = _sp(_dot(ue, wue_ref[...]) + _dot(uv, wuv_ref[...])
            + _dot(u_ref[...], wuu_ref[...]) + bu1_ref[...])
    h = _sp(_dot(h, wu2_ref[...]) + bu2_ref[...])
    out_ref[...] = _sp(_dot(h, wu3_ref[...]) + bu3_ref[...])


def _u_mlp(ue_sum, gcnt, uv_sum, ncnt, u, wue, wuv, wuu, bu1, wu2, bu2,
           wu3, bu3):
    ud = u.shape[1]
    full = lambda shape: pl.BlockSpec(shape, lambda: (0,) * len(shape))
    return pl.pallas_call(
        _u_body,
        in_specs=[
            full((_G, _F)), full((_G, 1)), full((_G, _F)), full((_G, 1)),
            full((_G, ud)), full((_F, _F)), full((_F, _F)), full((ud, _F)),
            full((1, _F)), full((_F, _F)), full((1, _F)), full((_F, _F)),
            full((1, _F)),
        ],
        out_specs=full((_G, _F)),
        out_shape=jax.ShapeDtypeStruct((_G, _F), jnp.float32),
    )(ue_sum, gcnt, uv_sum, ncnt, u, wue, wuv, wuu, bu1, wu2, bu2, wu3, bu3)


def _head_ij_body(r_ref, e_ref, bb_ref, u_ref, wc_ref, wd_ref, b1_ref,
                  w2_ref, b2_ref, w3_ref, b3_ref, out_ref):
    ohs = _quarter_onehots(bb_ref[0])
    u1 = _dot(u_ref[...], wd_ref[...])
    uterm = jnp.concatenate([_dot(oh, u1) for oh in ohs], axis=1)
    r4 = r_ref[...]
    rterm = jnp.concatenate([r4[:, 16 * j:16 * j + 4] for j in range(4)],
                            axis=1)
    h = rterm + _dot(e_ref[...], wc_ref[...]) + uterm + b1_ref[...]
    h = _sp(h)
    h = _sp(_dot(h, w2_ref[...]) + b2_ref[...])
    out_ref[...] = _sp(_dot(h, w3_ref[...]) + b3_ref[...])


def _head_ij(r4, e4, bb4, u, wc_blk, wd, b1t, w2_blk, b2t, w3_blk, b3t):
    return pl.pallas_call(
        _head_ij_body,
        grid=(_NTE4,),
        in_specs=[
            pl.BlockSpec((_TE4, 64), lambda i: (i, 0)),
            pl.BlockSpec((_TE4, 128), lambda i: (i, 0)),
            pl.BlockSpec((1, _TE4, 4), lambda i: (i, 0, 0)),
            pl.BlockSpec((_G, _F), lambda i: (0, 0)),
            pl.BlockSpec((128, 16), lambda i: (0, 0)),
            pl.BlockSpec((_F, 4), lambda i: (0, 0)),
            pl.BlockSpec((1, 16), lambda i: (0, 0)),
            pl.BlockSpec((16, 16), lambda i: (0, 0)),
            pl.BlockSpec((1, 16), lambda i: (0, 0)),
            pl.BlockSpec((16, 16), lambda i: (0, 0)),
            pl.BlockSpec((1, 16), lambda i: (0, 0)),
        ],
        out_specs=pl.BlockSpec((_TE4, 16), lambda i: (i, 0)),
        out_shape=jax.ShapeDtypeStruct((_E // 4, 16), jnp.float32),
    )(r4, e4, bb4, u, wc_blk, wd, b1t, w2_blk, b2t, w3_blk, b3t)


def _head_ii_body(v_ref, b_ref, u_ref, wa_ref, wb_ref, b1_ref,
                  w2_ref, b2_ref, w3_ref, b3_ref, out_ref):
    ohs = _quarter_onehots(b_ref[0])
    u1 = _dot(u_ref[...], wb_ref[...])
    uterm = jnp.concatenate([_dot(oh, u1) for oh in ohs], axis=1)
    h = _sp(_dot(v_ref[...], wa_ref[...]) + uterm + b1_ref[...])
    h = _sp(_dot(h, w2_ref[...]) + b2_ref[...])
    out_ref[...] = _sp(_dot(h, w3_ref[...]) + b3_ref[...])


def _head_ii(v4, nb4, u, wa_blk, wb, b1t, w2_blk, b2t, w3_blk, b3t):
    return pl.pallas_call(
        _head_ii_body,
        grid=(_NTN4,),
        in_specs=[
            pl.BlockSpec((_TN4, 128), lambda i: (i, 0)),
            pl.BlockSpec((1, _TN4, 4), lambda i: (i, 0, 0)),
            pl.BlockSpec((_G, _F), lambda i: (0, 0)),
            pl.BlockSpec((128, 16), lambda i: (0, 0)),
            pl.BlockSpec((_F, 4), lambda i: (0, 0)),
            pl.BlockSpec((1, 16), lambda i: (0, 0)),
            pl.BlockSpec((16, 16), lambda i: (0, 0)),
            pl.BlockSpec((1, 16), lambda i: (0, 0)),
            pl.BlockSpec((16, 16), lambda i: (0, 0)),
            pl.BlockSpec((1, 16), lambda i: (0, 0)),
        ],
        out_specs=pl.BlockSpec((_TN4, 16), lambda i: (i, 0)),
        out_shape=jax.ShapeDtypeStruct((_N // 4, 16), jnp.float32),
    )(v4, nb4, u, wa_blk, wb, b1t, w2_blk, b2t, w3_blk, b3t)


def kernel(x, edge_index, edge_attr, state, batch, bond_batch, params):
    src = edge_index[0]
    dst = edge_index[1]
    bb4 = bond_batch.reshape(_NTE4, _TE4, 4)
    nb4 = batch.reshape(_NTN4, _TN4, 4)

    names = ("embed", "core0", "core1")
    (Wj1, bj1), (Wj2, bj2), (Wj3, bj3) = params["head_ij"]
    wja_pad = jnp.pad(Wj1[:_F], ((0, 0), (0, 12)))
    wjb_pad = jnp.pad(Wj1[_F:2 * _F], ((0, 0), (0, 12)))

    cnt4 = None
    u = state
    ein4 = _p4(edge_attr)
    for b, name in enumerate(names):
        bp = params[name]
        (W1, b1), (W2, b2), (W3, b3) = bp["phi_e"]
        n_in = 128 if b == 0 else _F
        e_in = ein4.shape[1] // 4
        w1a = W1[:n_in]
        w1b = W1[n_in:2 * n_in]
        w1c = W1[2 * n_in:2 * n_in + e_in]
        w1d = W1[2 * n_in + e_in:]
        (Wv1, bv1), (Wv2, bv2), (Wv3, bv3) = bp["phi_v"]
        wv1a = Wv1[:n_in]
        wv1b = Wv1[n_in:n_in + _F]
        wv1c = Wv1[n_in + _F:]

        if b == 0:
            A, B, VX = _node_proj(x, w1a, w1b, wv1a)
            vx4 = _p4(VX)

        Gm = _gather_rows_add(A, B, src, dst, _F)
        e4, ue_sum, gcnt = _edge_mlp(
            _p4(Gm), ein4, bb4, u, _bd(w1c), w1d, _t4(b1), _bd(W2),
            _t4(b2), _bd(W3), _t4(b3))
        if cnt4 is None:
            es_part, cnt_part = _scatter_count_partials(
                e4.reshape(_E, _F), dst)
            cnt4 = cnt_part.reshape(2, _N // 4, 64)
        else:
            es_part = _scatter_partials(e4.reshape(_E, _F), dst)
        es4 = es_part.reshape(2, _N // 4, 128)

        if b < 2:
            W1n = params[names[b + 1]]["phi_e"][0][0]
            Wv1n = params[names[b + 1]]["phi_v"][0][0]
            extras_w = [_bd(W1n[:_F]), _bd(W1n[_F:2 * _F]), _bd(Wv1n[:_F])]
            head_ws = []
        else:
            extras_w = [_bd(wja_pad), _bd(wjb_pad)]
            (Wi1, bi1), (Wi2, bi2), (Wi3, bi3) = params["head_ii"]
            head_ws = [_bd(Wi1[:_F]), Wi1[_F:], _t4(bi1), _bd(Wi2),
                       _t4(bi2), _bd(Wi3), _t4(bi3)]
        (Wu1, bu1), (Wu2, bu2), (Wu3, bu3) = bp["phi_u"]
        u_ws = [Wu1[:_F], Wu1[_F:2 * _F], Wu1[2 * _F:],
                bu1.reshape(1, -1), Wu2, bu2.reshape(1, -1), Wu3,
                bu3.reshape(1, -1)]
        outs = _node_mlp(
            vx4, es4, cnt4, nb4, u, _bd(wv1b), wv1c, _t4(bv1), _bd(Wv2),
            _t4(bv2), _bd(Wv3), _t4(bv3), ue_sum, gcnt, u_ws, extras_w,
            head_ws)
        v4, u = outs[:2]
        if b < 2:
            A = outs[2].reshape(_N, _F)
            B = outs[3].reshape(_N, _F)
            vx4 = outs[4]
        else:
            P = outs[2].reshape(_N, 16)
            Q = outs[3].reshape(_N, 16)
            ham_ii4 = outs[4]
        ein4 = e4

    R = _gather_rows_add(P, Q, src, dst, 16)
    ham_ij4 = _head_ij(R.reshape(_E // 4, 64), ein4, bb4, u,
                       _bd(Wj1[2 * _F:3 * _F]), Wj1[3 * _F:], _t4(bj1),
                       _bd(Wj2), _t4(bj2), _bd(Wj3), _t4(bj3))

    return (ham_ii4.reshape(_N, 4), ham_ij4.reshape(_E, 4), edge_index.T)


# speedup vs baseline: 7.4260x; 1.0103x over previous
import functools

import jax
import jax.numpy as jnp
from jax import lax
from jax.experimental import pallas as pl
from jax.experimental.pallas import tpu as pltpu
from jax.experimental.pallas import tpu_sc as plsc

_N = 10000
_E = 320000
_G = 64
_F = 32

_TE4 = 2000
_NTE4 = _E // 4 // _TE4
_TN4 = 2500
_NTN4 = _N // 4 // _TN4

_NW = 32
_ZR = _N // 16


def _sp(x):
    return jnp.maximum(x, 0.0) + jnp.log(1.0 + jnp.exp(-jnp.abs(x)))


def _mesh():
    return plsc.VectorSubcoreMesh(core_axis_name="c", subcore_axis_name="s")


_SC_PARAMS = pltpu.CompilerParams(use_tc_tiling_on_sc=False)


_CM = 400
_NM = _E // _NW // _CM
_GS = 80


@functools.lru_cache(maxsize=None)
def _gather_add_fn(width):
    nsl = width // 16

    def body(a_hbm, b_hbm, src_hbm, dst_hbm, out_hbm,
             si0, si1, di0, di1, ra0, ra1, rb0, rb1,
             sI0, sI1, sG0, sG1, sW0, sW1):
        si = (si0, si1)
        di = (di0, di1)
        ra = (ra0, ra1)
        rb = (rb0, rb1)
        sI = (sI0, sI1)
        sG = (sG0, sG1)
        sW = (sW0, sW1)
        w = lax.axis_index("s") * 2 + lax.axis_index("c")
        base = w * (_E // _NW)

        def issue_gathers(q):
            for j in range(_CM // _GS):
                sl = pl.ds(j * _GS, _GS)
                pltpu.async_copy(a_hbm.at[si[q].at[sl]], ra[q].at[sl], sG[q])
                pltpu.async_copy(b_hbm.at[di[q].at[sl]], rb[q].at[sl], sG[q])

        def wait_gathers(q):
            for j in range(_CM // _GS):
                sl = pl.ds(j * _GS, _GS)
                pltpu.make_async_copy(
                    a_hbm.at[si[q].at[sl]], ra[q].at[sl], sG[q]).wait()
                pltpu.make_async_copy(
                    b_hbm.at[di[q].at[sl]], rb[q].at[sl], sG[q]).wait()

        i0a = pltpu.async_copy(src_hbm.at[pl.ds(base, _CM)], si[0], sI[0])
        i0b = pltpu.async_copy(dst_hbm.at[pl.ds(base, _CM)], di[0], sI[0])
        i0a.wait()
        i0b.wait()
        issue_gathers(0)
        pltpu.async_copy(src_hbm.at[pl.ds(base + _CM, _CM)], si[1], sI[1])
        pltpu.async_copy(dst_hbm.at[pl.ds(base + _CM, _CM)], di[1], sI[1])

        def outer(mm, carry):
            for p in range(2):
                m = mm * 2 + p

                @pl.when(m < _NM)
                def _():
                    off = base + m * _CM
                    wait_gathers(p)

                    @pl.when(m + 2 < _NM)
                    def _():
                        off2 = off + 2 * _CM
                        pltpu.async_copy(
                            src_hbm.at[pl.ds(off2, _CM)], si[p], sI[p])
                        pltpu.async_copy(
                            dst_hbm.at[pl.ds(off2, _CM)], di[p], sI[p])

                    @pl.when(m + 1 < _NM)
                    def _():
                        pltpu.make_async_copy(
                            src_hbm.at[pl.ds(0, _CM)], si[1 - p],
                            sI[1 - p]).wait()
                        pltpu.make_async_copy(
                            dst_hbm.at[pl.ds(0, _CM)], di[1 - p],
                            sI[1 - p]).wait()

                        @pl.when(m >= 1)
                        def _():
                            pltpu.make_async_copy(
                                ra[1 - p], out_hbm.at[pl.ds(0, _CM)],
                                sW[1 - p]).wait()

                        issue_gathers(1 - p)

                    def add8(it, c2):
                        r0 = it * 8
                        for rr in range(8):
                            for jj in range(nsl):
                                s2 = pl.ds(jj * 16, 16)
                                ra[p][r0 + rr, s2] = (
                                    ra[p][r0 + rr, s2] + rb[p][r0 + rr, s2])
                        return c2

                    lax.fori_loop(0, _CM // 8, add8, 0)
                    pltpu.async_copy(ra[p], out_hbm.at[pl.ds(off, _CM)], sW[p])

            return carry

        lax.fori_loop(0, (_NM + 1) // 2, outer, 0)
        pltpu.make_async_copy(ra[1], out_hbm.at[pl.ds(0, _CM)], sW[1]).wait()
        pltpu.make_async_copy(ra[0], out_hbm.at[pl.ds(0, _CM)], sW[0]).wait()

    return pl.kernel(
        body,
        mesh=_mesh(),
        compiler_params=_SC_PARAMS,
        out_type=jax.ShapeDtypeStruct((_E, width), jnp.float32),
        scratch_types=[
            pltpu.VMEM((_CM,), jnp.int32),
            pltpu.VMEM((_CM,), jnp.int32),
            pltpu.VMEM((_CM,), jnp.int32),
            pltpu.VMEM((_CM,), jnp.int32),
            pltpu.VMEM((_CM, width), jnp.float32),
            pltpu.VMEM((_CM, width), jnp.float32),
            pltpu.VMEM((_CM, width), jnp.float32),
            pltpu.VMEM((_CM, width), jnp.float32),
            pltpu.SemaphoreType.DMA,
            pltpu.SemaphoreType.DMA,
            pltpu.SemaphoreType.DMA,
            pltpu.SemaphoreType.DMA,
            pltpu.SemaphoreType.DMA,
            pltpu.SemaphoreType.DMA,
        ],
    )


@functools.lru_cache(maxsize=None)
def _scatter_fn(with_count):
    width = _F

    def body(*args):
        if with_count:
            (e_hbm, dst_hbm, out_hbm, cout_hbm,
             si0, si1, eb0, eb1, ones, zb, zb2, acc, cacc,
             sI0, sI1, sE0, sE1, sO) = args
        else:
            (e_hbm, dst_hbm, out_hbm,
             si0, si1, eb0, eb1, zb, acc,
             sI0, sI1, sE0, sE1, sO) = args
        si = (si0, si1)
        eb = (eb0, eb1)
        sI = (sI0, sI1)
        sE = (sE0, sE1)
        cid = lax.axis_index("c")
        sid = lax.axis_index("s")
        w = sid * 2 + cid
        base = w * (_E // _NW)

        def zrow(i, carry):
            for jj in range(width // 16):
                zb[i, pl.ds(jj * 16, 16)] = jnp.zeros((16,), jnp.float32)
            if with_count:
                zb2[i, pl.ds(0, 16)] = jnp.zeros((16,), jnp.float32)
            return carry

        lax.fori_loop(0, _ZR, zrow, 0)
        if with_count:
            def orow(i, carry):
                ones[i, pl.ds(0, 16)] = jnp.ones((16,), jnp.float32)
                return carry

            lax.fori_loop(0, _CM, orow, 0)
        pltpu.async_copy(dst_hbm.at[pl.ds(base, _CM)], si[0], sI[0])
        pltpu.async_copy(e_hbm.at[pl.ds(base, _CM)], eb[0], sE[0])
        pltpu.sync_copy(zb, acc.at[pl.ds(sid * _ZR, _ZR)])
        if with_count:
            pltpu.sync_copy(zb2, cacc.at[pl.ds(sid * _ZR, _ZR)])
        plsc.subcore_barrier()

        def outer(mm, carry):
            for p in range(2):
                m = mm * 2 + p

                @pl.when(m < _NM)
                def _():
                    off = base + m * _CM
                    pltpu.make_async_copy(
                        dst_hbm.at[pl.ds(0, _CM)], si[p], sI[p]).wait()
                    pltpu.make_async_copy(
                        e_hbm.at[pl.ds(0, _CM)], eb[p], sE[p]).wait()

                    @pl.when(m + 1 < _NM)
                    def _():
                        off2 = off + _CM
                        pltpu.async_copy(
                            dst_hbm.at[pl.ds(off2, _CM)], si[1 - p], sI[1 - p])
                        pltpu.async_copy(
                            e_hbm.at[pl.ds(off2, _CM)], eb[1 - p], sE[1 - p])

                    pltpu.sync_copy(eb[p], acc.at[si[p]], add=True)
                    if with_count:
                        pltpu.sync_copy(ones, cacc.at[si[p]], add=True)

            return carry

        lax.fori_loop(0, (_NM + 1) // 2, outer, 0)
        plsc.subcore_barrier()
        rows = pl.ds(sid * _ZR, _ZR)
        cps = [pltpu.async_copy(acc.at[rows], out_hbm.at[cid, rows], sO)]
        if with_count:
            cps.append(pltpu.async_copy(cacc.at[rows], cout_hbm.at[cid, rows],
                                        sO))
        for cp in cps:
            cp.wait()

    out_type = [jax.ShapeDtypeStruct((2, _N, width), jnp.float32)]
    scratch = [
        pltpu.VMEM((_CM,), jnp.int32),
        pltpu.VMEM((_CM,), jnp.int32),
        pltpu.VMEM((_CM, width), jnp.float32),
        pltpu.VMEM((_CM, width), jnp.float32),
    ]
    if with_count:
        out_type.append(jax.ShapeDtypeStruct((2, _N, 16), jnp.float32))
        scratch.append(pltpu.VMEM((_CM, 16), jnp.float32))
    scratch.append(pltpu.VMEM((_ZR, width), jnp.float32))
    if with_count:
        scratch.append(pltpu.VMEM((_ZR, 16), jnp.float32))
    scratch.append(pltpu.VMEM_SHARED((_N, width), jnp.float32))
    if with_count:
        scratch.append(pltpu.VMEM_SHARED((_N, 16), jnp.float32))
    scratch += [pltpu.SemaphoreType.DMA] * 5
    return pl.kernel(
        body,
        mesh=_mesh(),
        compiler_params=_SC_PARAMS,
        out_type=out_type if with_count else out_type[0],
        scratch_types=scratch,
    )


def _gather_rows_add(a, b, src, dst, width):
    return _gather_add_fn(width)(a, b, src, dst)


def _scatter_partials(e, dst):
    return _scatter_fn(False)(e, dst)


def _scatter_count_partials(e, dst):
    return _scatter_fn(True)(e, dst)


def _dot(a, b):
    return jnp.dot(a, b, preferred_element_type=jnp.float32)


def _dg0(a, b):
    return lax.dot_general(a, b, (((0,), (0,)), ((), ())),
                           preferred_element_type=jnp.float32)


def _p4(a):
    return a.reshape(a.shape[0] // 4, 4 * a.shape[1])


def _bd(w):
    return jnp.kron(jnp.eye(4, dtype=w.dtype), w)


def _t4(b):
    return jnp.tile(b.reshape(1, -1), (1, 4))


def _quarter_onehots(bb):
    iota = lax.broadcasted_iota(jnp.int32, (1, _G), 1)
    return [(bb[:, j:j + 1] == iota).astype(jnp.float32) for j in range(4)]


def _node_proj_body(x_ref, wa_ref, wb_ref, wc_ref, a_ref, b_ref, c_ref):
    x = x_ref[...]
    a_ref[...] = _dot(x, wa_ref[...])
    b_ref[...] = _dot(x, wb_ref[...])
    c_ref[...] = _dot(x, wc_ref[...])


def _node_proj(x, w1a, w1b, wv1a):
    d = x.shape[1]
    tn = 1000
    return pl.pallas_call(
        _node_proj_body,
        grid=(_N // tn,),
        in_specs=[
            pl.BlockSpec((tn, d), lambda i: (i, 0)),
            pl.BlockSpec((d, _F), lambda i: (0, 0)),
            pl.BlockSpec((d, _F), lambda i: (0, 0)),
            pl.BlockSpec((d, _F), lambda i: (0, 0)),
        ],
        out_specs=[pl.BlockSpec((tn, _F), lambda i: (i, 0))] * 3,
        out_shape=[jax.ShapeDtypeStruct((_N, _F), jnp.float32)] * 3,
    )(x, w1a, w1b, wv1a)


def _edge_body(g_ref, ea_ref, bb_ref, u_ref, w1c_ref, w1d_ref, b1_ref,
               w2_ref, b2_ref, w3_ref, b3_ref, e_ref, ue_ref, gc_ref):
    i = pl.program_id(0)
    ohs = _quarter_onehots(bb_ref[0])
    u1 = _dot(u_ref[...], w1d_ref[...])
    uterm = jnp.concatenate([_dot(oh, u1) for oh in ohs], axis=1)
    h = g_ref[...] + _dot(ea_ref[...], w1c_ref[...]) + uterm + b1_ref[...]
    h = _sp(h)
    h = _sp(_dot(h, w2_ref[...]) + b2_ref[...])
    h = _sp(_dot(h, w3_ref[...]) + b3_ref[...])
    e_ref[...] = h
    ones = jnp.ones((h.shape[0], 1), jnp.float32)
    part = sum(_dg0(ohs[j], h[:, 32 * j:32 * j + 32]) for j in range(4))
    cnt = sum(_dg0(ohs[j], ones) for j in range(4))

    @pl.when(i == 0)
    def _():
        ue_ref[...] = part
        gc_ref[...] = cnt

    @pl.when(i > 0)
    def _():
        ue_ref[...] = ue_ref[...] + part
        gc_ref[...] = gc_ref[...] + cnt


def _edge_mlp(g4, ea4, bb4, u, w1c_blk, w1d, b1t, w2_blk, b2t, w3_blk, b3t):
    ein4 = ea4.shape[1]
    ud = u.shape[1]
    return pl.pallas_call(
        _edge_body,
        grid=(_NTE4,),
        in_specs=[
            pl.BlockSpec((_TE4, 128), lambda i: (i, 0)),
            pl.BlockSpec((_TE4, ein4), lambda i: (i, 0)),
            pl.BlockSpec((1, _TE4, 4), lambda i: (i, 0, 0)),
            pl.BlockSpec((_G, ud), lambda i: (0, 0)),
            pl.BlockSpec((ein4, 128), lambda i: (0, 0)),
            pl.BlockSpec((ud, _F), lambda i: (0, 0)),
            pl.BlockSpec((1, 128), lambda i: (0, 0)),
            pl.BlockSpec((128, 128), lambda i: (0, 0)),
            pl.BlockSpec((1, 128), lambda i: (0, 0)),
            pl.BlockSpec((128, 128), lambda i: (0, 0)),
            pl.BlockSpec((1, 128), lambda i: (0, 0)),
        ],
        out_specs=[
            pl.BlockSpec((_TE4, 128), lambda i: (i, 0)),
            pl.BlockSpec((_G, _F), lambda i: (0, 0)),
            pl.BlockSpec((_G, 1), lambda i: (0, 0)),
        ],
        out_shape=[
            jax.ShapeDtypeStruct((_E // 4, 128), jnp.float32),
            jax.ShapeDtypeStruct((_G, _F), jnp.float32),
            jax.ShapeDtypeStruct((_G, 1), jnp.float32),
        ],
    )(g4, ea4, bb4, u, w1c_blk, w1d, b1t, w2_blk, b2t, w3_blk, b3t)


def _make_node_body(n_extra, with_head):
    def body(*refs):
        (vx_ref, es_ref, cn_ref, b_ref, u_ref, wv1b_ref, wv1c_ref, bv1_ref,
         wv2_ref, bv2_ref, wv3_ref, bv3_ref, ue_ref, gc_ref, wue_ref,
         wuv_ref, wuu_ref, bu1_ref, wu2_ref, bu2_ref, wu3_ref,
         bu3_ref) = refs[:22]
        k = 22
        ew_refs = refs[k:k + n_extra]
        k += n_extra
        if with_head:
            (wia_ref, wib_ref, bi1_ref, wi2_ref, bi2_ref, wi3_ref,
             bi3_ref) = refs[k:k + 7]
            k += 7
        v_ref = refs[k]
        unew_ref = refs[k + 1]
        ex_refs = refs[k + 2:k + 2 + n_extra]
        if with_head:
            hii_ref = refs[k + 2 + n_extra]
        es = es_ref[0] + es_ref[1]
        cn = cn_ref[0] + cn_ref[1]
        div = jnp.concatenate(
            [jnp.broadcast_to(cn[:, 16 * j:16 * j + 1], (_TN4, 32))
             for j in range(4)], axis=1)
        emean = es / jnp.maximum(div, 1.0)
        ohs = _quarter_onehots(b_ref[0])
        u = u_ref[...]
        u1 = _dot(u, wv1c_ref[...])
        uterm = jnp.concatenate([_dot(oh, u1) for oh in ohs], axis=1)
        h = vx_ref[...] + _dot(emean, wv1b_ref[...]) + uterm + bv1_ref[...]
        h = _sp(h)
        h = _sp(_dot(h, wv2_ref[...]) + bv2_ref[...])
        v = _sp(_dot(h, wv3_ref[...]) + bv3_ref[...])
        v_ref[...] = v
        for ew, ex in zip(ew_refs, ex_refs):
            ex[...] = _dot(v, ew[...])
        ones = jnp.ones((v.shape[0], 1), jnp.float32)
        uv_sum = sum(_dg0(ohs[j], v[:, 32 * j:32 * j + 32]) for j in range(4))
        ncnt = sum(_dg0(ohs[j], ones) for j in range(4))
        ue = ue_ref[...] / jnp.maximum(gc_ref[...], 1.0)
        uv = uv_sum / jnp.maximum(ncnt, 1.0)
        hu = _sp(_dot(ue, wue_ref[...]) + _dot(uv, wuv_ref[...])
                 + _dot(u, wuu_ref[...]) + bu1_ref[...])
        hu = _sp(_dot(hu, wu2_ref[...]) + bu2_ref[...])
        u_new = _sp(_dot(hu, wu3_ref[...]) + bu3_ref[...])
        unew_ref[...] = u_new
        if with_head:
            u1h = _dot(u_new, wib_ref[...])
            uth = jnp.concatenate([_dot(oh, u1h) for oh in ohs], axis=1)
            hh = _sp(_dot(v, wia_ref[...]) + uth + bi1_ref[...])
            hh = _sp(_dot(hh, wi2_ref[...]) + bi2_ref[...])
            hii_ref[...] = _sp(_dot(hh, wi3_ref[...]) + bi3_ref[...])

    return body


def _node_mlp(vx4, es4, cn4, nb4, u, wv1b_blk, wv1c, bv1t, wv2_blk, bv2t,
              wv3_blk, bv3t, ue_sum, gcnt, u_ws, extras_w, head_ws):
    ud = u.shape[1]
    full = lambda r, c: pl.BlockSpec((r, c), lambda i: (0, 0))
    in_specs = [
        pl.BlockSpec((_TN4, 128), lambda i: (i, 0)),
        pl.BlockSpec((2, _TN4, 128), lambda i: (0, i, 0)),
        pl.BlockSpec((2, _TN4, 64), lambda i: (0, i, 0)),
        pl.BlockSpec((1, _TN4, 4), lambda i: (i, 0, 0)),
        full(_G, ud), full(128, 128), full(ud, _F), full(1, 128),
        full(128, 128), full(1, 128), full(128, 128), full(1, 128),
        full(_G, _F), full(_G, 1),
    ] + [full(*w.shape) for w in u_ws] \
      + [full(128, w.shape[1]) for w in extras_w] \
      + [full(*w.shape) for w in head_ws]
    out_specs = [
        pl.BlockSpec((_TN4, 128), lambda i: (i, 0)),
        pl.BlockSpec((_G, _F), lambda i: (0, 0)),
    ] + [pl.BlockSpec((_TN4, w.shape[1]), lambda i: (i, 0))
         for w in extras_w]
    out_shape = [
        jax.ShapeDtypeStruct((_N // 4, 128), jnp.float32),
        jax.ShapeDtypeStruct((_G, _F), jnp.float32),
    ] + [jax.ShapeDtypeStruct((_N // 4, w.shape[1]), jnp.float32)
         for w in extras_w]
    if head_ws:
        out_specs.append(pl.BlockSpec((_TN4, 16), lambda i: (i, 0)))
        out_shape.append(jax.ShapeDtypeStruct((_N // 4, 16), jnp.float32))
    return pl.pallas_call(
        _make_node_body(len(extras_w), bool(head_ws)),
        grid=(_NTN4,),
        in_specs=in_specs,
        out_specs=out_specs,
        out_shape=out_shape,
    )(vx4, es4, cn4, nb4, u, wv1b_blk, wv1c, bv1t, wv2_blk, bv2t, wv3_blk,
      bv3t, ue_sum, gcnt, *u_ws, *extras_w, *head_ws)


def _u_body(ue_ref, gc_ref, uv_ref, nc_ref, u_ref, wue_ref, wuv_ref, wuu_ref,
            bu1_ref, wu2_ref, bu2_ref, wu3_ref, bu3_ref, out_ref):
    ue = ue_ref[...] / jnp.maximum(gc_ref[...], 1.0)
    uv = uv_ref[...] / jnp.maximum(nc_ref[...], 1.0)
    h = _sp(_dot(ue, wue_ref[...]) + _dot(uv, wuv_ref[...])
            + _dot(u_ref[...], wuu_ref[...]) + bu1_ref[...])
    h = _sp(_dot(h, wu2_ref[...]) + bu2_ref[...])
    out_ref[...] = _sp(_dot(h, wu3_ref[...]) + bu3_ref[...])


def _u_mlp(ue_sum, gcnt, uv_sum, ncnt, u, wue, wuv, wuu, bu1, wu2, bu2,
           wu3, bu3):
    ud = u.shape[1]
    full = lambda shape: pl.BlockSpec(shape, lambda: (0,) * len(shape))
    return pl.pallas_call(
        _u_body,
        in_specs=[
            full((_G, _F)), full((_G, 1)), full((_G, _F)), full((_G, 1)),
            full((_G, ud)), full((_F, _F)), full((_F, _F)), full((ud, _F)),
            full((1, _F)), full((_F, _F)), full((1, _F)), full((_F, _F)),
            full((1, _F)),
        ],
        out_specs=full((_G, _F)),
        out_shape=jax.ShapeDtypeStruct((_G, _F), jnp.float32),
    )(ue_sum, gcnt, uv_sum, ncnt, u, wue, wuv, wuu, bu1, wu2, bu2, wu3, bu3)


def _head_ij_body(r_ref, e_ref, bb_ref, u_ref, wc_ref, wd_ref, b1_ref,
                  w2_ref, b2_ref, w3_ref, b3_ref, out_ref):
    ohs = _quarter_onehots(bb_ref[0])
    u1 = _dot(u_ref[...], wd_ref[...])
    uterm = jnp.concatenate([_dot(oh, u1) for oh in ohs], axis=1)
    r4 = r_ref[...]
    rterm = jnp.concatenate([r4[:, 16 * j:16 * j + 4] for j in range(4)],
                            axis=1)
    h = rterm + _dot(e_ref[...], wc_ref[...]) + uterm + b1_ref[...]
    h = _sp(h)
    h = _sp(_dot(h, w2_ref[...]) + b2_ref[...])
    out_ref[...] = _sp(_dot(h, w3_ref[...]) + b3_ref[...])


def _head_ij(r4, e4, bb4, u, wc_blk, wd, b1t, w2_blk, b2t, w3_blk, b3t):
    return pl.pallas_call(
        _head_ij_body,
        grid=(_NTE4,),
        in_specs=[
            pl.BlockSpec((_TE4, 64), lambda i: (i, 0)),
            pl.BlockSpec((_TE4, 128), lambda i: (i, 0)),
            pl.BlockSpec((1, _TE4, 4), lambda i: (i, 0, 0)),
            pl.BlockSpec((_G, _F), lambda i: (0, 0)),
            pl.BlockSpec((128, 16), lambda i: (0, 0)),
            pl.BlockSpec((_F, 4), lambda i: (0, 0)),
            pl.BlockSpec((1, 16), lambda i: (0, 0)),
            pl.BlockSpec((16, 16), lambda i: (0, 0)),
            pl.BlockSpec((1, 16), lambda i: (0, 0)),
            pl.BlockSpec((16, 16), lambda i: (0, 0)),
            pl.BlockSpec((1, 16), lambda i: (0, 0)),
        ],
        out_specs=pl.BlockSpec((_TE4, 16), lambda i: (i, 0)),
        out_shape=jax.ShapeDtypeStruct((_E // 4, 16), jnp.float32),
    )(r4, e4, bb4, u, wc_blk, wd, b1t, w2_blk, b2t, w3_blk, b3t)


def _head_ii_body(v_ref, b_ref, u_ref, wa_ref, wb_ref, b1_ref,
                  w2_ref, b2_ref, w3_ref, b3_ref, out_ref):
    ohs = _quarter_onehots(b_ref[0])
    u1 = _dot(u_ref[...], wb_ref[...])
    uterm = jnp.concatenate([_dot(oh, u1) for oh in ohs], axis=1)
    h = _sp(_dot(v_ref[...], wa_ref[...]) + uterm + b1_ref[...])
    h = _sp(_dot(h, w2_ref[...]) + b2_ref[...])
    out_ref[...] = _sp(_dot(h, w3_ref[...]) + b3_ref[...])


def _head_ii(v4, nb4, u, wa_blk, wb, b1t, w2_blk, b2t, w3_blk, b3t):
    return pl.pallas_call(
        _head_ii_body,
        grid=(_NTN4,),
        in_specs=[
            pl.BlockSpec((_TN4, 128), lambda i: (i, 0)),
            pl.BlockSpec((1, _TN4, 4), lambda i: (i, 0, 0)),
            pl.BlockSpec((_G, _F), lambda i: (0, 0)),
            pl.BlockSpec((128, 16), lambda i: (0, 0)),
            pl.BlockSpec((_F, 4), lambda i: (0, 0)),
            pl.BlockSpec((1, 16), lambda i: (0, 0)),
            pl.BlockSpec((16, 16), lambda i: (0, 0)),
            pl.BlockSpec((1, 16), lambda i: (0, 0)),
            pl.BlockSpec((16, 16), lambda i: (0, 0)),
            pl.BlockSpec((1, 16), lambda i: (0, 0)),
        ],
        out_specs=pl.BlockSpec((_TN4, 16), lambda i: (i, 0)),
        out_shape=jax.ShapeDtypeStruct((_N // 4, 16), jnp.float32),
    )(v4, nb4, u, wa_blk, wb, b1t, w2_blk, b2t, w3_blk, b3t)


def kernel(x, edge_index, edge_attr, state, batch, bond_batch, params):
    src = edge_index[0]
    dst = edge_index[1]
    bb4 = bond_batch.reshape(_NTE4, _TE4, 4)
    nb4 = batch.reshape(_NTN4, _TN4, 4)

    names = ("embed", "core0", "core1")
    (Wj1, bj1), (Wj2, bj2), (Wj3, bj3) = params["head_ij"]
    wja_pad = jnp.pad(Wj1[:_F], ((0, 0), (0, 12)))
    wjb_pad = jnp.pad(Wj1[_F:2 * _F], ((0, 0), (0, 12)))

    cnt4 = None
    u = state
    ein4 = _p4(edge_attr)
    for b, name in enumerate(names):
        bp = params[name]
        (W1, b1), (W2, b2), (W3, b3) = bp["phi_e"]
        n_in = 128 if b == 0 else _F
        e_in = ein4.shape[1] // 4
        w1a = W1[:n_in]
        w1b = W1[n_in:2 * n_in]
        w1c = W1[2 * n_in:2 * n_in + e_in]
        w1d = W1[2 * n_in + e_in:]
        (Wv1, bv1), (Wv2, bv2), (Wv3, bv3) = bp["phi_v"]
        wv1a = Wv1[:n_in]
        wv1b = Wv1[n_in:n_in + _F]
        wv1c = Wv1[n_in + _F:]

        if b == 0:
            A, B, VX = _node_proj(x, w1a, w1b, wv1a)
            vx4 = _p4(VX)

        Gm = _gather_rows_add(A, B, src, dst, _F)
        e4, ue_sum, gcnt = _edge_mlp(
            _p4(Gm), ein4, bb4, u, _bd(w1c), w1d, _t4(b1), _bd(W2),
            _t4(b2), _bd(W3), _t4(b3))
        if cnt4 is None:
            es_part, cnt_part = _scatter_count_partials(
                e4.reshape(_E, _F), dst)
            cnt4 = cnt_part.reshape(2, _N // 4, 64)
        else:
            es_part = _scatter_partials(e4.reshape(_E, _F), dst)
        es4 = es_part.reshape(2, _N // 4, 128)

        if b < 2:
            W1n = params[names[b + 1]]["phi_e"][0][0]
            Wv1n = params[names[b + 1]]["phi_v"][0][0]
            extras_w = [_bd(W1n[:_F]), _bd(W1n[_F:2 * _F]), _bd(Wv1n[:_F])]
            head_ws = []
        else:
            extras_w = [_bd(wja_pad), _bd(wjb_pad)]
            (Wi1, bi1), (Wi2, bi2), (Wi3, bi3) = params["head_ii"]
            head_ws = [_bd(Wi1[:_F]), Wi1[_F:], _t4(bi1), _bd(Wi2),
                       _t4(bi2), _bd(Wi3), _t4(bi3)]
        (Wu1, bu1), (Wu2, bu2), (Wu3, bu3) = bp["phi_u"]
        u_ws = [Wu1[:_F], Wu1[_F:2 * _F], Wu1[2 * _F:],
                bu1.reshape(1, -1), Wu2, bu2.reshape(1, -1), Wu3,
                bu3.reshape(1, -1)]
        outs = _node_mlp(
            vx4, es4, cnt4, nb4, u, _bd(wv1b), wv1c, _t4(bv1), _bd(Wv2),
            _t4(bv2), _bd(Wv3), _t4(bv3), ue_sum, gcnt, u_ws, extras_w,
            head_ws)
        v4, u = outs[:2]
        if b < 2:
            A = outs[2].reshape(_N, _F)
            B = outs[3].reshape(_N, _F)
            vx4 = outs[4]
        else:
            P = outs[2].reshape(_N, 16)
            Q = outs[3].reshape(_N, 16)
            ham_ii4 = outs[4]
        ein4 = e4

    R = _gather_rows_add(P, Q, src, dst, 16)
    ham_ij4 = _head_ij(R.reshape(_E // 4, 64), ein4, bb4, u,
                       _bd(Wj1[2 * _F:3 * _F]), Wj1[3 * _F:], _t4(bj1),
                       _bd(Wj2), _t4(bj2), _bd(Wj3), _t4(bj3))

    return (ham_ii4.reshape(_N, 4), ham_ij4.reshape(_E, 4), edge_index.T)
